# Initial kernel scaffold; baseline (speedup 1.0000x reference)
import jax
import jax.numpy as jnp
from jax.experimental import pallas as pl


def kernel(graph_data, edge_index, edge_attr, clinical_data, batch, W1, a_src1, a_dst1, We1, a_e1, b1, W2, a_src2, a_dst2, We2, a_e2, b2, Wc1, bc1, Wc2, bc2, Wg1, bg1, Wg2, bg2, Wcls, bcls):
    raise NotImplementedError("write your pallas kernel here")


# speedup vs baseline: 1.4695x; 1.4695x over previous
import jax
import jax.numpy as jnp
from jax.experimental import pallas as pl

N = 10000
B = 64


def _identity_kernel(x_ref, o_ref):
    o_ref[...] = x_ref[...]


def _gat_conv(x, src, dst, edge_attr, W, a_src, a_dst, We, a_e, b):
    h = x @ W
    ae = edge_attr @ (We @ a_e)
    alpha = (h @ a_src)[src] + (h @ a_dst)[dst] + ae
    alpha = jax.nn.leaky_relu(alpha, 0.2)
    ex = jnp.exp(alpha)
    denom = jax.ops.segment_sum(ex, dst, num_segments=N)
    num = jax.ops.segment_sum(h[src] * ex[:, None], dst, num_segments=N)
    return num / (denom[:, None] + 1e-16) + b


def kernel(graph_data, edge_index, edge_attr, clinical_data, batch, W1, a_src1, a_dst1, We1, a_e1, b1, W2, a_src2, a_dst2, We2, a_e2, b2, Wc1, bc1, Wc2, bc2, Wg1, bg1, Wg2, bg2, Wcls, bcls):
    src, dst = edge_index[0], edge_index[1]
    x = jax.nn.relu(_gat_conv(graph_data, src, dst, edge_attr, W1, a_src1, a_dst1, We1, a_e1, b1))
    x = jax.nn.relu(_gat_conv(x, src, dst, edge_attr, W2, a_src2, a_dst2, We2, a_e2, b2))
    cnt = jnp.clip(jax.ops.segment_sum(jnp.ones((N,), jnp.float32), batch, num_segments=B), 1.0, None)
    gmean = jax.ops.segment_sum(x, batch, num_segments=B) / cnt[:, None]
    gmax = jax.ops.segment_max(x, batch, num_segments=B)
    gmax = jnp.where(jnp.isfinite(gmax), gmax, 0.0)
    emb_graph = jnp.concatenate([gmean, gmax], axis=1)
    hc = jax.nn.relu(clinical_data @ Wc1 + bc1)
    emb_clinical = hc @ Wc2 + bc2
    emb_graph = emb_graph / jnp.maximum(jnp.linalg.norm(emb_graph, axis=1, keepdims=True), 1e-12)
    emb_clinical = emb_clinical / jnp.maximum(jnp.linalg.norm(emb_clinical, axis=1, keepdims=True), 1e-12)
    cat = jnp.concatenate([emb_graph, emb_clinical], axis=1)
    gate = jax.nn.sigmoid(jax.nn.relu(cat @ Wg1 + bg1) @ Wg2 + bg2)
    comb = jnp.concatenate([emb_graph, emb_clinical * gate], axis=-1)
    out = comb @ Wcls + bcls
    return pl.pallas_call(
        _identity_kernel,
        out_shape=jax.ShapeDtypeStruct(out.shape, out.dtype),
    )(out)


# speedup vs baseline: 9.0556x; 6.1622x over previous
import functools

import jax
import jax.numpy as jnp
from jax import lax
from jax.experimental import pallas as pl
from jax.experimental.pallas import tpu as pltpu
from jax.experimental.pallas import tpu_sc as plsc

N = 10000
E = 160000
F = 256
FE = 16
H = 256
B = 64
CLIN = 32
NCLS = 4

HH = H // 2
NCORE = 2
NSUB = 16
LN = 16
CK = 128
NCHUNKS = E // CK
ITERS = -(-NCHUNKS // NSUB)
RPT = N // NSUB
SEG_PT = B // (NSUB // 2)

_f32 = jnp.float32


def _node_proj_body(x_ref, wfull_ref, asrc_ref, adst_ref, whalf_ref,
                    h_ref, ss_ref, sd_ref):
    x = x_ref[...]
    h_ref[...] = jnp.dot(x, whalf_ref[...], preferred_element_type=_f32)

    @pl.when(pl.program_id(1) == 0)
    def _():
        wv_s = jnp.dot(wfull_ref[...], asrc_ref[...],
                       preferred_element_type=_f32)
        wv_d = jnp.dot(wfull_ref[...], adst_ref[...],
                       preferred_element_type=_f32)
        ss_ref[...] = jnp.dot(x, wv_s, preferred_element_type=_f32)[:, None]
        sd_ref[...] = jnp.dot(x, wv_d, preferred_element_type=_f32)[:, None]


def _node_proj(x, w, a_src, a_dst):
    nb = 10
    blk = N // nb
    return pl.pallas_call(
        _node_proj_body,
        grid=(nb, NCORE),
        in_specs=[
            pl.BlockSpec((blk, F), lambda i, j: (i, 0)),
            pl.BlockSpec((F, H), lambda i, j: (0, 0)),
            pl.BlockSpec((H,), lambda i, j: (0,)),
            pl.BlockSpec((H,), lambda i, j: (0,)),
            pl.BlockSpec((F, HH), lambda i, j: (0, j)),
        ],
        out_specs=[
            pl.BlockSpec((blk, HH), lambda i, j: (j * nb + i, 0)),
            pl.BlockSpec((blk, 1), lambda i, j: (i, 0)),
            pl.BlockSpec((blk, 1), lambda i, j: (i, 0)),
        ],
        out_shape=[
            jax.ShapeDtypeStruct((2 * N, HH), _f32),
            jax.ShapeDtypeStruct((N, 1), _f32),
            jax.ShapeDtypeStruct((N, 1), _f32),
        ],
    )(x, w, a_src, a_dst, w)


def _edge_proj_body(ea_ref, we1_ref, ae1v_ref, we2_ref, ae2v_ref,
                    o1_ref, o2_ref):
    ea = ea_ref[...]
    w1 = jnp.dot(we1_ref[...], ae1v_ref[...], preferred_element_type=_f32)
    w2 = jnp.dot(we2_ref[...], ae2v_ref[...], preferred_element_type=_f32)
    o1_ref[...] = jnp.dot(ea, w1, preferred_element_type=_f32)[:, None]
    o2_ref[...] = jnp.dot(ea, w2, preferred_element_type=_f32)[:, None]


def _edge_proj(edge_attr, we1, a_e1, we2, a_e2):
    nb = 80
    blk = E // nb
    return pl.pallas_call(
        _edge_proj_body,
        grid=(nb,),
        in_specs=[
            pl.BlockSpec((blk, FE), lambda i: (i, 0)),
            pl.BlockSpec((FE, H), lambda i: (0, 0)),
            pl.BlockSpec((H,), lambda i: (0,)),
            pl.BlockSpec((FE, H), lambda i: (0, 0)),
            pl.BlockSpec((H,), lambda i: (0,)),
        ],
        out_specs=[
            pl.BlockSpec((blk, 1), lambda i: (i, 0)),
            pl.BlockSpec((blk, 1), lambda i: (i, 0)),
        ],
        out_shape=[
            jax.ShapeDtypeStruct((E, 1), _f32),
            jax.ShapeDtypeStruct((E, 1), _f32),
        ],
    )(edge_attr, we1, a_e1, we2, a_e2)


def _mid_proj_body(numlo_ref, numhi_ref, den_ref, b_ref, wfull_ref,
                   asrc_ref, adst_ref, whalf_ref, h_ref, ss_ref, sd_ref):
    num = jnp.concatenate([numlo_ref[...], numhi_ref[...]], axis=1)
    den = den_ref[...][0, 0][:, None]
    x = jnp.maximum(num / (den + 1e-16) + b_ref[...], 0.0)
    h_ref[...] = jnp.dot(x, whalf_ref[...], preferred_element_type=_f32)

    @pl.when(pl.program_id(1) == 0)
    def _():
        wv_s = jnp.dot(wfull_ref[...], asrc_ref[...],
                       preferred_element_type=_f32)
        wv_d = jnp.dot(wfull_ref[...], adst_ref[...],
                       preferred_element_type=_f32)
        ss_ref[...] = jnp.dot(x, wv_s, preferred_element_type=_f32)[:, None]
        sd_ref[...] = jnp.dot(x, wv_d, preferred_element_type=_f32)[:, None]


def _mid_proj(num, den, b, w, a_src, a_dst):
    nb = 10
    blk = N // nb
    return pl.pallas_call(
        _mid_proj_body,
        grid=(nb, NCORE),
        in_specs=[
            pl.BlockSpec((blk, HH), lambda i, j: (i, 0)),
            pl.BlockSpec((blk, HH), lambda i, j: (nb + i, 0)),
            pl.BlockSpec((1, 1, blk), lambda i, j: (i, 0, 0)),
            pl.BlockSpec((H,), lambda i, j: (0,)),
            pl.BlockSpec((H, H), lambda i, j: (0, 0)),
            pl.BlockSpec((H,), lambda i, j: (0,)),
            pl.BlockSpec((H,), lambda i, j: (0,)),
            pl.BlockSpec((H, HH), lambda i, j: (0, j)),
        ],
        out_specs=[
            pl.BlockSpec((blk, HH), lambda i, j: (j * nb + i, 0)),
            pl.BlockSpec((blk, 1), lambda i, j: (i, 0)),
            pl.BlockSpec((blk, 1), lambda i, j: (i, 0)),
        ],
        out_shape=[
            jax.ShapeDtypeStruct((2 * N, HH), _f32),
            jax.ShapeDtypeStruct((N, 1), _f32),
            jax.ShapeDtypeStruct((N, 1), _f32),
        ],
    )(num, num, den.reshape(nb, 1, blk), b, w, a_src, a_dst, w)


def _sc_message_body(h_hbm, ssrc_hbm, sdst_hbm, ae_hbm, src_hbm, dst_hbm,
                     num_hbm, den_hbm,
                     ssrc_v, sdst_v, srcidx_v, dstidx_v, ae_v, ex_v, rows_v,
                     den_z, acc_sh, den_sh):
    c = lax.axis_index("c")
    s = lax.axis_index("s")

    pltpu.sync_copy(ssrc_hbm, ssrc_v)
    pltpu.sync_copy(sdst_hbm, sdst_v)

    zeros = jnp.zeros((LN,), _f32)

    def _zero_den(i, _):
        den_z[pl.ds(i * LN, LN)] = zeros
        return 0
    lax.fori_loop(0, N // LN, _zero_den, 0)

    def _zero_rows(i, _):
        for k in range(HH // LN):
            rows_v[i, pl.ds(k * LN, LN)] = zeros
        return 0
    lax.fori_loop(0, CK, _zero_rows, 0)

    def _zero_acc(i, _):
        ci = i * NSUB + s

        @pl.when(ci < N // LN)
        def _():
            pltpu.sync_copy(rows_v.at[pl.ds(0, LN)],
                            acc_sh.at[pl.ds(ci * LN, LN)])
        return 0
    lax.fori_loop(0, -(-(N // LN) // NSUB), _zero_acc, 0)

    @pl.when(s == 0)
    def _():
        pltpu.sync_copy(den_z, den_sh)

    plsc.subcore_barrier()

    coff = c * N

    def _chunk(i, _):
        ci = i * NSUB + s

        @pl.when(ci < NCHUNKS)
        def _():
            base = ci * CK
            pltpu.sync_copy(src_hbm.at[pl.ds(base, CK)], srcidx_v)
            pltpu.sync_copy(dst_hbm.at[pl.ds(base, CK)], dstidx_v)
            pltpu.sync_copy(ae_hbm.at[pl.ds(base, CK)], ae_v)

            for j in range(CK // LN):
                sl = pl.ds(j * LN, LN)
                sv = srcidx_v[sl]
                dv = dstidx_v[sl]
                a = (plsc.load_gather(ssrc_v, [sv])
                     + plsc.load_gather(sdst_v, [dv])
                     + ae_v[sl])
                a = jnp.maximum(a, 0.2 * a)
                ex_v[sl] = jnp.exp(a)
                srcidx_v[sl] = sv + coff

            pltpu.sync_copy(h_hbm.at[srcidx_v], rows_v)

            def _scale(g, _):
                exv = ex_v[pl.ds(g * LN, LN)]
                for l in range(LN):
                    exj = exv[l]
                    jj = g * LN + l
                    for k in range(HH // LN):
                        slk = pl.ds(k * LN, LN)
                        rows_v[jj, slk] = rows_v[jj, slk] * exj
                return 0
            lax.fori_loop(0, CK // LN, _scale, 0)

            pltpu.sync_copy(rows_v, acc_sh.at[dstidx_v], add=True)
            pltpu.sync_copy(ex_v, den_sh.at[dstidx_v], add=True)
        return 0

    lax.fori_loop(0, ITERS, _chunk, 0)

    plsc.subcore_barrier()

    def _out_copy(i, _):
        ci = i * NSUB + s

        @pl.when(ci < N // LN)
        def _():
            pltpu.sync_copy(acc_sh.at[pl.ds(ci * LN, LN)],
                            num_hbm.at[pl.ds(coff + ci * LN, LN)])
        return 0
    lax.fori_loop(0, -(-(N // LN) // NSUB), _out_copy, 0)

    @pl.when((s == 0) & (c == 0))
    def _():
        pltpu.sync_copy(den_sh, den_hbm)


_sc_message = functools.partial(
    pl.kernel,
    _sc_message_body,
    out_type=(
        jax.ShapeDtypeStruct((2 * N, HH), _f32),
        jax.ShapeDtypeStruct((N,), _f32),
    ),
    mesh=plsc.VectorSubcoreMesh(core_axis_name="c", subcore_axis_name="s",
                                num_cores=NCORE, num_subcores=NSUB),
    compiler_params=pltpu.CompilerParams(needs_layout_passes=False),
    scratch_types=[
        pltpu.VMEM((N,), _f32),
        pltpu.VMEM((N,), _f32),
        pltpu.VMEM((CK,), jnp.int32),
        pltpu.VMEM((CK,), jnp.int32),
        pltpu.VMEM((CK,), _f32),
        pltpu.VMEM((CK,), _f32),
        pltpu.VMEM((CK, HH), _f32),
        pltpu.VMEM((N,), _f32),
        pltpu.VMEM_SHARED((N, HH), _f32),
        pltpu.VMEM_SHARED((N,), _f32),
    ],
)()


NPT = 632
NPT_LAST = N - (NSUB - 1) * NPT


def _sc_pool_body(num_hbm, den_hbm, b2_hbm, batch_hbm,
                  gsum_hbm, gmax_hbm,
                  x_v, den_v, batch_v, b2_v, sum_v, max_v, mg_v, pool_sh):
    c = lax.axis_index("c")
    s = lax.axis_index("s")
    base = s * NPT
    is_last = s == NSUB - 1

    @pl.when(is_last)
    def _():
        pltpu.sync_copy(num_hbm.at[pl.ds(c * N + base, NPT_LAST)],
                        x_v.at[pl.ds(0, NPT_LAST)])
        pltpu.sync_copy(den_hbm.at[pl.ds(base, NPT_LAST)],
                        den_v.at[pl.ds(0, NPT_LAST)])
        pltpu.sync_copy(batch_hbm.at[pl.ds(base, NPT_LAST)],
                        batch_v.at[pl.ds(0, NPT_LAST)])

    @pl.when(jnp.logical_not(is_last))
    def _():
        pltpu.sync_copy(num_hbm.at[pl.ds(c * N + base, NPT)], x_v)
        pltpu.sync_copy(den_hbm.at[pl.ds(base, NPT)], den_v)
        pltpu.sync_copy(batch_hbm.at[pl.ds(base, NPT)], batch_v)

    pltpu.sync_copy(b2_hbm, b2_v)

    zeros = jnp.zeros((LN,), _f32)

    def _zero_acc(i, _):
        for k in range(HH // LN):
            sum_v[i, pl.ds(k * LN, LN)] = zeros
            max_v[i, pl.ds(k * LN, LN)] = zeros
        return 0
    lax.fori_loop(0, B, _zero_acc, 0)

    b2c = [b2_v[pl.ds(c * HH + k * LN, LN)] for k in range(HH // LN)]

    sz = jnp.where(is_last, NPT_LAST, NPT)
    ngroups = jnp.where(is_last, NPT_LAST // LN, NPT // LN)

    def _do_node(n, b, rec):
        for k in range(HH // LN):
            slk = pl.ds(k * LN, LN)
            xa = jnp.maximum(x_v[n, slk] * rec + b2c[k], 0.0)
            sum_v[b, slk] = sum_v[b, slk] + xa
            max_v[b, slk] = jnp.maximum(max_v[b, slk], xa)

    def _node(g, _):
        @pl.when(g < ngroups)
        def _():
            bv = batch_v[pl.ds(g * LN, LN)]
            dv = den_v[pl.ds(g * LN, LN)]
            recv = 1.0 / (dv + 1e-16)
            for l in range(LN):
                _do_node(g * LN + l, bv[l], recv[l])
        return 0
    lax.fori_loop(0, NPT // LN, _node, 0)

    bv = batch_v[pl.ds(sz - LN, LN)]
    dv = den_v[pl.ds(sz - LN, LN)]
    recv = 1.0 / (dv + 1e-16)
    for l in range(LN // 2, LN):
        _do_node(sz - LN + l, bv[l], recv[l])

    for acc_v, out_hbm, is_max in ((sum_v, gsum_hbm, False),
                                   (max_v, gmax_hbm, True)):
        pltpu.sync_copy(acc_v, pool_sh.at[s])
        plsc.subcore_barrier()

        @pl.when(s < NSUB // 2)
        def _():
            for t in range(NSUB):
                pltpu.sync_copy(pool_sh.at[t].at[pl.ds(s * SEG_PT, SEG_PT)],
                                mg_v.at[t])

            def _merge_row(r, _):
                for k in range(HH // LN):
                    slk = pl.ds(k * LN, LN)
                    v = mg_v[0, r, slk]
                    for t in range(1, NSUB):
                        if is_max:
                            v = jnp.maximum(v, mg_v[t, r, slk])
                        else:
                            v = v + mg_v[t, r, slk]
                    sum_v[r, slk] = v
                return 0
            lax.fori_loop(0, SEG_PT, _merge_row, 0)
            pltpu.sync_copy(sum_v.at[pl.ds(0, SEG_PT)],
                            out_hbm.at[pl.ds(c * B + s * SEG_PT, SEG_PT)])
        plsc.subcore_barrier()


_sc_pool = functools.partial(
    pl.kernel,
    _sc_pool_body,
    out_type=(
        jax.ShapeDtypeStruct((NCORE * B, HH), _f32),
        jax.ShapeDtypeStruct((NCORE * B, HH), _f32),
    ),
    mesh=plsc.VectorSubcoreMesh(core_axis_name="c", subcore_axis_name="s",
                                num_cores=NCORE, num_subcores=NSUB),
    scratch_types=[
        pltpu.VMEM((NPT, HH), _f32),
        pltpu.VMEM((NPT,), _f32),
        pltpu.VMEM((NPT,), jnp.int32),
        pltpu.VMEM((H,), _f32),
        pltpu.VMEM((B, HH), _f32),
        pltpu.VMEM((B, HH), _f32),
        pltpu.VMEM((NSUB, SEG_PT, HH), _f32),
        pltpu.VMEM_SHARED((NSUB, B, HH), _f32),
    ],
)()


def _final_body(gsum_ref, gmax_ref, batch_ref, clin_ref, wc1_ref, bc1_ref,
                wc2_ref, bc2_ref, wg1_ref, bg1_ref, wg2_ref, bg2_ref,
                wcls_ref, bcls_ref, o_ref):
    bt = batch_ref[...]
    eq = bt[:, None] == lax.broadcasted_iota(jnp.int32, (N, B), 1)
    cnt = jnp.sum(jnp.where(eq, 1.0, 0.0), axis=0)
    cnt = jnp.clip(cnt, 1.0, None)

    gsum = jnp.concatenate([gsum_ref[...][:B], gsum_ref[...][B:]], axis=1)
    gmax = jnp.concatenate([gmax_ref[...][:B], gmax_ref[...][B:]], axis=1)
    gmean = gsum / cnt[:, None]
    emb_g = jnp.concatenate([gmean, gmax], axis=1)

    hc = jnp.maximum(
        jnp.dot(clin_ref[...], wc1_ref[...], preferred_element_type=_f32)
        + bc1_ref[...], 0.0)
    emb_c = (jnp.dot(hc, wc2_ref[...], preferred_element_type=_f32)
             + bc2_ref[...])

    ng = jnp.sqrt(jnp.sum(emb_g * emb_g, axis=1, keepdims=True))
    emb_g = emb_g / jnp.maximum(ng, 1e-12)
    nc = jnp.sqrt(jnp.sum(emb_c * emb_c, axis=1, keepdims=True))
    emb_c = emb_c / jnp.maximum(nc, 1e-12)

    cat = jnp.concatenate([emb_g, emb_c], axis=1)
    g1 = jnp.maximum(
        jnp.dot(cat, wg1_ref[...], preferred_element_type=_f32)
        + bg1_ref[...], 0.0)
    gate = jax.nn.sigmoid(
        jnp.dot(g1, wg2_ref[...], preferred_element_type=_f32) + bg2_ref[...])
    comb = jnp.concatenate([emb_g, emb_c * gate], axis=-1)
    o_ref[...] = (jnp.dot(comb, wcls_ref[...], preferred_element_type=_f32)
                  + bcls_ref[...])


def _final(gsum, gmax, batch, clinical, wc1, bc1, wc2, bc2,
           wg1, bg1, wg2, bg2, wcls, bcls):
    return pl.pallas_call(
        _final_body,
        out_shape=jax.ShapeDtypeStruct((B, NCLS), _f32),
    )(gsum, gmax, batch, clinical, wc1, bc1, wc2, bc2,
      wg1, bg1, wg2, bg2, wcls, bcls)


def kernel(graph_data, edge_index, edge_attr, clinical_data, batch,
           W1, a_src1, a_dst1, We1, a_e1, b1,
           W2, a_src2, a_dst2, We2, a_e2, b2,
           Wc1, bc1, Wc2, bc2, Wg1, bg1, Wg2, bg2, Wcls, bcls):
    src = edge_index[0]
    dst = edge_index[1]

    h1, s1s, s1d = _node_proj(graph_data, W1, a_src1, a_dst1)
    ae1, ae2 = _edge_proj(edge_attr, We1, a_e1, We2, a_e2)

    num1, den1 = _sc_message(h1, jnp.squeeze(s1s, 1), jnp.squeeze(s1d, 1),
                             jnp.squeeze(ae1, 1), src, dst)

    h2, s2s, s2d = _mid_proj(num1, den1, b1, W2, a_src2, a_dst2)

    num2, den2 = _sc_message(h2, jnp.squeeze(s2s, 1), jnp.squeeze(s2d, 1),
                             jnp.squeeze(ae2, 1), src, dst)

    gsum, gmax = _sc_pool(num2, den2, b2, batch)

    return _final(gsum, gmax, batch, clinical_data,
                  Wc1, bc1, Wc2, bc2, Wg1, bg1, Wg2, bg2, Wcls, bcls)


# speedup vs baseline: 11.8773x; 1.3116x over previous
import functools

import jax
import jax.numpy as jnp
from jax import lax
from jax.experimental import pallas as pl
from jax.experimental.pallas import tpu as pltpu
from jax.experimental.pallas import tpu_sc as plsc

N = 10000
E = 160000
F = 256
FE = 16
H = 256
B = 64
CLIN = 32
NCLS = 4

HH = H // 2
NCORE = 2
NSUB = 16
LN = 16
CK = 64
NCHUNKS = E // CK
ITERS = -(-NCHUNKS // NSUB)
RPT = N // NSUB
SEG_PT = B // (NSUB // 2)

_f32 = jnp.float32


def _node_proj_body(x_ref, wfull_ref, asrc_ref, adst_ref, whalf_ref,
                    h_ref, ss_ref, sd_ref):
    x = x_ref[...]
    h_ref[...] = jnp.dot(x, whalf_ref[...], preferred_element_type=_f32)

    @pl.when(pl.program_id(1) == 0)
    def _():
        wv_s = jnp.dot(wfull_ref[...], asrc_ref[...],
                       preferred_element_type=_f32)
        wv_d = jnp.dot(wfull_ref[...], adst_ref[...],
                       preferred_element_type=_f32)
        ss_ref[...] = jnp.dot(x, wv_s, preferred_element_type=_f32)[:, None]
        sd_ref[...] = jnp.dot(x, wv_d, preferred_element_type=_f32)[:, None]


def _node_proj(x, w, a_src, a_dst):
    nb = 10
    blk = N // nb
    return pl.pallas_call(
        _node_proj_body,
        grid=(nb, NCORE),
        in_specs=[
            pl.BlockSpec((blk, F), lambda i, j: (i, 0)),
            pl.BlockSpec((F, H), lambda i, j: (0, 0)),
            pl.BlockSpec((H,), lambda i, j: (0,)),
            pl.BlockSpec((H,), lambda i, j: (0,)),
            pl.BlockSpec((F, HH), lambda i, j: (0, j)),
        ],
        out_specs=[
            pl.BlockSpec((blk, HH), lambda i, j: (j * nb + i, 0)),
            pl.BlockSpec((blk, 1), lambda i, j: (i, 0)),
            pl.BlockSpec((blk, 1), lambda i, j: (i, 0)),
        ],
        out_shape=[
            jax.ShapeDtypeStruct((2 * N, HH), _f32),
            jax.ShapeDtypeStruct((N, 1), _f32),
            jax.ShapeDtypeStruct((N, 1), _f32),
        ],
    )(x, w, a_src, a_dst, w)


def _edge_proj_body(ea_ref, we1_ref, ae1v_ref, we2_ref, ae2v_ref,
                    o1_ref, o2_ref):
    ea = ea_ref[...]
    w1 = jnp.dot(we1_ref[...], ae1v_ref[...], preferred_element_type=_f32)
    w2 = jnp.dot(we2_ref[...], ae2v_ref[...], preferred_element_type=_f32)
    o1_ref[...] = jnp.dot(ea, w1, preferred_element_type=_f32)[:, None]
    o2_ref[...] = jnp.dot(ea, w2, preferred_element_type=_f32)[:, None]


def _edge_proj(edge_attr, we1, a_e1, we2, a_e2):
    nb = 80
    blk = E // nb
    return pl.pallas_call(
        _edge_proj_body,
        grid=(nb,),
        in_specs=[
            pl.BlockSpec((blk, FE), lambda i: (i, 0)),
            pl.BlockSpec((FE, H), lambda i: (0, 0)),
            pl.BlockSpec((H,), lambda i: (0,)),
            pl.BlockSpec((FE, H), lambda i: (0, 0)),
            pl.BlockSpec((H,), lambda i: (0,)),
        ],
        out_specs=[
            pl.BlockSpec((blk, 1), lambda i: (i, 0)),
            pl.BlockSpec((blk, 1), lambda i: (i, 0)),
        ],
        out_shape=[
            jax.ShapeDtypeStruct((E, 1), _f32),
            jax.ShapeDtypeStruct((E, 1), _f32),
        ],
    )(edge_attr, we1, a_e1, we2, a_e2)


def _mid_proj_body(numlo_ref, numhi_ref, den_ref, b_ref, wfull_ref,
                   asrc_ref, adst_ref, whalf_ref, h_ref, ss_ref, sd_ref):
    num = jnp.concatenate([numlo_ref[...], numhi_ref[...]], axis=1)
    den = den_ref[...][0, 0][:, None]
    x = jnp.maximum(num / (den + 1e-16) + b_ref[...], 0.0)
    h_ref[...] = jnp.dot(x, whalf_ref[...], preferred_element_type=_f32)

    @pl.when(pl.program_id(1) == 0)
    def _():
        wv_s = jnp.dot(wfull_ref[...], asrc_ref[...],
                       preferred_element_type=_f32)
        wv_d = jnp.dot(wfull_ref[...], adst_ref[...],
                       preferred_element_type=_f32)
        ss_ref[...] = jnp.dot(x, wv_s, preferred_element_type=_f32)[:, None]
        sd_ref[...] = jnp.dot(x, wv_d, preferred_element_type=_f32)[:, None]


def _mid_proj(num, den, b, w, a_src, a_dst):
    nb = 10
    blk = N // nb
    return pl.pallas_call(
        _mid_proj_body,
        grid=(nb, NCORE),
        in_specs=[
            pl.BlockSpec((blk, HH), lambda i, j: (i, 0)),
            pl.BlockSpec((blk, HH), lambda i, j: (nb + i, 0)),
            pl.BlockSpec((1, 1, blk), lambda i, j: (i, 0, 0)),
            pl.BlockSpec((H,), lambda i, j: (0,)),
            pl.BlockSpec((H, H), lambda i, j: (0, 0)),
            pl.BlockSpec((H,), lambda i, j: (0,)),
            pl.BlockSpec((H,), lambda i, j: (0,)),
            pl.BlockSpec((H, HH), lambda i, j: (0, j)),
        ],
        out_specs=[
            pl.BlockSpec((blk, HH), lambda i, j: (j * nb + i, 0)),
            pl.BlockSpec((blk, 1), lambda i, j: (i, 0)),
            pl.BlockSpec((blk, 1), lambda i, j: (i, 0)),
        ],
        out_shape=[
            jax.ShapeDtypeStruct((2 * N, HH), _f32),
            jax.ShapeDtypeStruct((N, 1), _f32),
            jax.ShapeDtypeStruct((N, 1), _f32),
        ],
    )(num, num, den.reshape(nb, 1, blk), b, w, a_src, a_dst, w)


NSTEPS = -(-NCHUNKS // NSUB) * NSUB // NSUB
NPAIRS = (NSTEPS + 1) // 2


def _sc_message_body(h_hbm, ssrc_hbm, sdst_hbm, ae_hbm, src_hbm, dst_hbm,
                     num_hbm, den_hbm,
                     ssrc_v, sdst_v,
                     srcA, dstA, aeA, exA, rowsA,
                     srcB, dstB, aeB, exB, rowsB,
                     den_z, ssem, gsem, wsem, acc_sh, den_sh):
    c = lax.axis_index("c")
    s = lax.axis_index("s")

    pltpu.sync_copy(ssrc_hbm, ssrc_v)
    pltpu.sync_copy(sdst_hbm, sdst_v)

    zeros = jnp.zeros((LN,), _f32)

    def _zero_den(i, _):
        den_z[pl.ds(i * LN, LN)] = zeros
        return 0
    lax.fori_loop(0, N // LN, _zero_den, 0)

    def _zero_rows(i, _):
        for k in range(HH // LN):
            rowsA[i, pl.ds(k * LN, LN)] = zeros
        return 0
    lax.fori_loop(0, LN, _zero_rows, 0)

    def _zero_acc(i, _):
        ci = i * NSUB + s

        @pl.when(ci < N // LN)
        def _():
            pltpu.sync_copy(rowsA.at[pl.ds(0, LN)],
                            acc_sh.at[pl.ds(ci * LN, LN)])
        return 0
    lax.fori_loop(0, -(-(N // LN) // NSUB), _zero_acc, 0)

    @pl.when(s == 0)
    def _():
        pltpu.sync_copy(den_z, den_sh)

    plsc.subcore_barrier()

    coff = c * N


    def _stage(ci, srcv, dstv, aev):
        base = jnp.minimum(ci, NCHUNKS - 1) * CK
        return (pltpu.async_copy(src_hbm.at[pl.ds(base, CK)], srcv, ssem),
                pltpu.async_copy(dst_hbm.at[pl.ds(base, CK)], dstv, ssem),
                pltpu.async_copy(ae_hbm.at[pl.ds(base, CK)], aev, ssem))

    def _wait_w(rowsv, dstv, exv):
        pltpu.make_async_copy(rowsv, acc_sh.at[dstv], wsem).wait()
        pltpu.make_async_copy(exv, den_sh.at[dstv], wsem).wait()

    def _fix_gather(srcv, dstv, rowsv):
        raws, dsts = [], []
        for j in range(CK // LN):
            sl = pl.ds(j * LN, LN)
            sv = srcv[sl]
            raws.append(sv)
            dsts.append(dstv[sl])
            srcv[sl] = sv + coff
        return raws, dsts, pltpu.async_copy(h_hbm.at[srcv], rowsv, gsem)

    def _ex_compute(ci, raws, dsts, aev, exv):
        vf = jnp.where(ci < NCHUNKS, 1.0, 0.0).astype(_f32)
        for j in range(CK // LN):
            sl = pl.ds(j * LN, LN)
            a = (plsc.load_gather(ssrc_v, [raws[j]])
                 + plsc.load_gather(sdst_v, [dsts[j]])
                 + aev[sl])
            a = jnp.maximum(a, 0.2 * a)
            exv[sl] = jnp.exp(a) * vf

    def _scale(rowsv, exv):
        def body(g, _):
            exg = exv[pl.ds(g * LN, LN)]
            for l in range(LN):
                e = exg[l]
                jj = g * LN + l
                for k in range(HH // LN):
                    slk = pl.ds(k * LN, LN)
                    rowsv[jj, slk] = rowsv[jj, slk] * e
            return 0
        lax.fori_loop(0, CK // LN, body, 0)

    def _scatter(rowsv, dstv, exv):
        pltpu.async_copy(rowsv, acc_sh.at[dstv], wsem, add=True)
        pltpu.async_copy(exv, den_sh.at[dstv], wsem, add=True)

    def _pair(i, _):
        ci0 = (2 * i) * NSUB + s
        ci1 = (2 * i + 1) * NSUB + s

        @pl.when(i > 0)
        def _():
            _wait_w(rowsA, dstA, exA)
            _wait_w(rowsB, dstB, exB)

        stA = _stage(ci0, srcA, dstA, aeA)
        stB = _stage(ci1, srcB, dstB, aeB)
        for d in stA:
            d.wait()
        rawsA, dstsA, gA = _fix_gather(srcA, dstA, rowsA)
        for d in stB:
            d.wait()
        rawsB, dstsB, gB = _fix_gather(srcB, dstB, rowsB)

        _ex_compute(ci0, rawsA, dstsA, aeA, exA)
        _ex_compute(ci1, rawsB, dstsB, aeB, exB)

        gA.wait()
        _scale(rowsA, exA)
        _scatter(rowsA, dstA, exA)

        gB.wait()
        _scale(rowsB, exB)
        _scatter(rowsB, dstB, exB)
        return 0

    lax.fori_loop(0, NPAIRS, _pair, 0)

    _wait_w(rowsA, dstA, exA)
    _wait_w(rowsB, dstB, exB)

    plsc.subcore_barrier()

    def _out_copy(i, _):
        ci = i * NSUB + s

        @pl.when(ci < N // LN)
        def _():
            pltpu.sync_copy(acc_sh.at[pl.ds(ci * LN, LN)],
                            num_hbm.at[pl.ds(coff + ci * LN, LN)])
        return 0
    lax.fori_loop(0, -(-(N // LN) // NSUB), _out_copy, 0)

    @pl.when((s == 0) & (c == 0))
    def _():
        pltpu.sync_copy(den_sh, den_hbm)


_sc_message = functools.partial(
    pl.kernel,
    _sc_message_body,
    out_type=(
        jax.ShapeDtypeStruct((2 * N, HH), _f32),
        jax.ShapeDtypeStruct((N,), _f32),
    ),
    mesh=plsc.VectorSubcoreMesh(core_axis_name="c", subcore_axis_name="s",
                                num_cores=NCORE, num_subcores=NSUB),
    compiler_params=pltpu.CompilerParams(needs_layout_passes=False),
    scratch_types=[
        pltpu.VMEM((N,), _f32),
        pltpu.VMEM((N,), _f32),
        pltpu.VMEM((CK,), jnp.int32),
        pltpu.VMEM((CK,), jnp.int32),
        pltpu.VMEM((CK,), _f32),
        pltpu.VMEM((CK,), _f32),
        pltpu.VMEM((CK, HH), _f32),
        pltpu.VMEM((CK,), jnp.int32),
        pltpu.VMEM((CK,), jnp.int32),
        pltpu.VMEM((CK,), _f32),
        pltpu.VMEM((CK,), _f32),
        pltpu.VMEM((CK, HH), _f32),
        pltpu.VMEM((N,), _f32),
        pltpu.SemaphoreType.DMA,
        pltpu.SemaphoreType.DMA,
        pltpu.SemaphoreType.DMA,
        pltpu.VMEM_SHARED((N, HH), _f32),
        pltpu.VMEM_SHARED((N,), _f32),
    ],
)()


NPT = 632
NPT_LAST = N - (NSUB - 1) * NPT


def _sc_pool_body(num_hbm, den_hbm, b2_hbm, batch_hbm,
                  gsum_hbm, gmax_hbm,
                  x_v, den_v, batch_v, b2_v, sum_v, max_v, mg_v, pool_sh):
    c = lax.axis_index("c")
    s = lax.axis_index("s")
    base = s * NPT
    is_last = s == NSUB - 1

    @pl.when(is_last)
    def _():
        pltpu.sync_copy(num_hbm.at[pl.ds(c * N + base, NPT_LAST)],
                        x_v.at[pl.ds(0, NPT_LAST)])
        pltpu.sync_copy(den_hbm.at[pl.ds(base, NPT_LAST)],
                        den_v.at[pl.ds(0, NPT_LAST)])
        pltpu.sync_copy(batch_hbm.at[pl.ds(base, NPT_LAST)],
                        batch_v.at[pl.ds(0, NPT_LAST)])

    @pl.when(jnp.logical_not(is_last))
    def _():
        pltpu.sync_copy(num_hbm.at[pl.ds(c * N + base, NPT)], x_v)
        pltpu.sync_copy(den_hbm.at[pl.ds(base, NPT)], den_v)
        pltpu.sync_copy(batch_hbm.at[pl.ds(base, NPT)], batch_v)

    pltpu.sync_copy(b2_hbm, b2_v)

    zeros = jnp.zeros((LN,), _f32)

    def _zero_acc(i, _):
        for k in range(HH // LN):
            sum_v[i, pl.ds(k * LN, LN)] = zeros
            max_v[i, pl.ds(k * LN, LN)] = zeros
        return 0
    lax.fori_loop(0, B, _zero_acc, 0)

    b2c = [b2_v[pl.ds(c * HH + k * LN, LN)] for k in range(HH // LN)]

    sz = jnp.where(is_last, NPT_LAST, NPT)
    ngroups = jnp.where(is_last, NPT_LAST // LN, NPT // LN)

    def _do_node(n, b, rec):
        for k in range(HH // LN):
            slk = pl.ds(k * LN, LN)
            xa = jnp.maximum(x_v[n, slk] * rec + b2c[k], 0.0)
            sum_v[b, slk] = sum_v[b, slk] + xa
            max_v[b, slk] = jnp.maximum(max_v[b, slk], xa)

    def _node(g, _):
        @pl.when(g < ngroups)
        def _():
            bv = batch_v[pl.ds(g * LN, LN)]
            dv = den_v[pl.ds(g * LN, LN)]
            recv = 1.0 / (dv + 1e-16)
            for l in range(LN):
                _do_node(g * LN + l, bv[l], recv[l])
        return 0
    lax.fori_loop(0, NPT // LN, _node, 0)

    bv = batch_v[pl.ds(sz - LN, LN)]
    dv = den_v[pl.ds(sz - LN, LN)]
    recv = 1.0 / (dv + 1e-16)
    for l in range(LN // 2, LN):
        _do_node(sz - LN + l, bv[l], recv[l])

    for acc_v, out_hbm, is_max in ((sum_v, gsum_hbm, False),
                                   (max_v, gmax_hbm, True)):
        pltpu.sync_copy(acc_v, pool_sh.at[s])
        plsc.subcore_barrier()

        @pl.when(s < NSUB // 2)
        def _():
            for t in range(NSUB):
                pltpu.sync_copy(pool_sh.at[t].at[pl.ds(s * SEG_PT, SEG_PT)],
                                mg_v.at[pl.ds(t * SEG_PT, SEG_PT)])

            def _merge_row(r, _):
                for k in range(HH // LN):
                    slk = pl.ds(k * LN, LN)
                    v = mg_v[r, slk]
                    for t in range(1, NSUB):
                        if is_max:
                            v = jnp.maximum(v, mg_v[t * SEG_PT + r, slk])
                        else:
                            v = v + mg_v[t * SEG_PT + r, slk]
                    sum_v[r, slk] = v
                return 0
            lax.fori_loop(0, SEG_PT, _merge_row, 0)
            pltpu.sync_copy(sum_v.at[pl.ds(0, SEG_PT)],
                            out_hbm.at[pl.ds(c * B + s * SEG_PT, SEG_PT)])
        plsc.subcore_barrier()


_sc_pool = functools.partial(
    pl.kernel,
    _sc_pool_body,
    out_type=(
        jax.ShapeDtypeStruct((NCORE * B, HH), _f32),
        jax.ShapeDtypeStruct((NCORE * B, HH), _f32),
    ),
    mesh=plsc.VectorSubcoreMesh(core_axis_name="c", subcore_axis_name="s",
                                num_cores=NCORE, num_subcores=NSUB),
    compiler_params=pltpu.CompilerParams(needs_layout_passes=False),
    scratch_types=[
        pltpu.VMEM((NPT, HH), _f32),
        pltpu.VMEM((NPT,), _f32),
        pltpu.VMEM((NPT,), jnp.int32),
        pltpu.VMEM((H,), _f32),
        pltpu.VMEM((B, HH), _f32),
        pltpu.VMEM((B, HH), _f32),
        pltpu.VMEM((NSUB * SEG_PT, HH), _f32),
        pltpu.VMEM_SHARED((NSUB, B, HH), _f32),
    ],
)()


def _final_body(gsum_ref, gmax_ref, batch_ref, clin_ref, wc1_ref, bc1_ref,
                wc2_ref, bc2_ref, wg1_ref, bg1_ref, wg2_ref, bg2_ref,
                wcls_ref, bcls_ref, o_ref):
    bt = batch_ref[...]
    eq = bt[:, None] == lax.broadcasted_iota(jnp.int32, (N, B), 1)
    cnt = jnp.sum(jnp.where(eq, 1.0, 0.0), axis=0)
    cnt = jnp.clip(cnt, 1.0, None)

    gsum = jnp.concatenate([gsum_ref[...][:B], gsum_ref[...][B:]], axis=1)
    gmax = jnp.concatenate([gmax_ref[...][:B], gmax_ref[...][B:]], axis=1)
    gmean = gsum / cnt[:, None]
    emb_g = jnp.concatenate([gmean, gmax], axis=1)

    hc = jnp.maximum(
        jnp.dot(clin_ref[...], wc1_ref[...], preferred_element_type=_f32)
        + bc1_ref[...], 0.0)
    emb_c = (jnp.dot(hc, wc2_ref[...], preferred_element_type=_f32)
             + bc2_ref[...])

    ng = jnp.sqrt(jnp.sum(emb_g * emb_g, axis=1, keepdims=True))
    emb_g = emb_g / jnp.maximum(ng, 1e-12)
    nc = jnp.sqrt(jnp.sum(emb_c * emb_c, axis=1, keepdims=True))
    emb_c = emb_c / jnp.maximum(nc, 1e-12)

    cat = jnp.concatenate([emb_g, emb_c], axis=1)
    g1 = jnp.maximum(
        jnp.dot(cat, wg1_ref[...], preferred_element_type=_f32)
        + bg1_ref[...], 0.0)
    gate = jax.nn.sigmoid(
        jnp.dot(g1, wg2_ref[...], preferred_element_type=_f32) + bg2_ref[...])
    comb = jnp.concatenate([emb_g, emb_c * gate], axis=-1)
    o_ref[...] = (jnp.dot(comb, wcls_ref[...], preferred_element_type=_f32)
                  + bcls_ref[...])


def _final(gsum, gmax, batch, clinical, wc1, bc1, wc2, bc2,
           wg1, bg1, wg2, bg2, wcls, bcls):
    return pl.pallas_call(
        _final_body,
        out_shape=jax.ShapeDtypeStruct((B, NCLS), _f32),
    )(gsum, gmax, batch, clinical, wc1, bc1, wc2, bc2,
      wg1, bg1, wg2, bg2, wcls, bcls)


def kernel(graph_data, edge_index, edge_attr, clinical_data, batch,
           W1, a_src1, a_dst1, We1, a_e1, b1,
           W2, a_src2, a_dst2, We2, a_e2, b2,
           Wc1, bc1, Wc2, bc2, Wg1, bg1, Wg2, bg2, Wcls, bcls):
    src = edge_index[0]
    dst = edge_index[1]

    h1, s1s, s1d = _node_proj(graph_data, W1, a_src1, a_dst1)
    ae1, ae2 = _edge_proj(edge_attr, We1, a_e1, We2, a_e2)

    num1, den1 = _sc_message(h1, jnp.squeeze(s1s, 1), jnp.squeeze(s1d, 1),
                             jnp.squeeze(ae1, 1), src, dst)

    h2, s2s, s2d = _mid_proj(num1, den1, b1, W2, a_src2, a_dst2)

    num2, den2 = _sc_message(h2, jnp.squeeze(s2s, 1), jnp.squeeze(s2d, 1),
                             jnp.squeeze(ae2, 1), src, dst)

    gsum, gmax = _sc_pool(num2, den2, b2, batch)

    return _final(gsum, gmax, batch, clinical_data,
                  Wc1, bc1, Wc2, bc2, Wg1, bg1, Wg2, bg2, Wcls, bcls)


# speedup vs baseline: 13.0511x; 1.0988x over previous
import functools

import jax
import jax.numpy as jnp
from jax import lax
from jax.experimental import pallas as pl
from jax.experimental.pallas import tpu as pltpu
from jax.experimental.pallas import tpu_sc as plsc

N = 10000
E = 160000
F = 256
FE = 16
H = 256
B = 64
CLIN = 32
NCLS = 4

HH = H // 2
NCORE = 2
NSUB = 16
LN = 16
CK = 80
NCHUNKS = E // CK
ITERS = -(-NCHUNKS // NSUB)
RPT = N // NSUB
SEG_PT = B // (NSUB // 2)

_f32 = jnp.float32


def _node_proj_body(x_ref, wfull_ref, asrc_ref, adst_ref, whalf_ref,
                    h_ref, ss_ref, sd_ref):
    x = x_ref[...]
    h_ref[...] = jnp.dot(x, whalf_ref[...], preferred_element_type=_f32)

    @pl.when(pl.program_id(1) == 0)
    def _():
        wv_s = jnp.dot(wfull_ref[...], asrc_ref[...],
                       preferred_element_type=_f32)
        wv_d = jnp.dot(wfull_ref[...], adst_ref[...],
                       preferred_element_type=_f32)
        ss_ref[...] = jnp.dot(x, wv_s, preferred_element_type=_f32)[:, None]
        sd_ref[...] = jnp.dot(x, wv_d, preferred_element_type=_f32)[:, None]


def _node_proj(x, w, a_src, a_dst):
    nb = 10
    blk = N // nb
    return pl.pallas_call(
        _node_proj_body,
        grid=(nb, NCORE),
        in_specs=[
            pl.BlockSpec((blk, F), lambda i, j: (i, 0)),
            pl.BlockSpec((F, H), lambda i, j: (0, 0)),
            pl.BlockSpec((H,), lambda i, j: (0,)),
            pl.BlockSpec((H,), lambda i, j: (0,)),
            pl.BlockSpec((F, HH), lambda i, j: (0, j)),
        ],
        out_specs=[
            pl.BlockSpec((blk, HH), lambda i, j: (j * nb + i, 0)),
            pl.BlockSpec((blk, 1), lambda i, j: (i, 0)),
            pl.BlockSpec((blk, 1), lambda i, j: (i, 0)),
        ],
        out_shape=[
            jax.ShapeDtypeStruct((2 * N, HH), _f32),
            jax.ShapeDtypeStruct((N, 1), _f32),
            jax.ShapeDtypeStruct((N, 1), _f32),
        ],
    )(x, w, a_src, a_dst, w)


def _edge_proj_body(ea_ref, we1_ref, ae1v_ref, we2_ref, ae2v_ref,
                    o1_ref, o2_ref):
    ea = ea_ref[...]
    w1 = jnp.dot(we1_ref[...], ae1v_ref[...], preferred_element_type=_f32)
    w2 = jnp.dot(we2_ref[...], ae2v_ref[...], preferred_element_type=_f32)
    o1_ref[...] = jnp.dot(ea, w1, preferred_element_type=_f32)[:, None]
    o2_ref[...] = jnp.dot(ea, w2, preferred_element_type=_f32)[:, None]


def _edge_proj(edge_attr, we1, a_e1, we2, a_e2):
    nb = 80
    blk = E // nb
    return pl.pallas_call(
        _edge_proj_body,
        grid=(nb,),
        in_specs=[
            pl.BlockSpec((blk, FE), lambda i: (i, 0)),
            pl.BlockSpec((FE, H), lambda i: (0, 0)),
            pl.BlockSpec((H,), lambda i: (0,)),
            pl.BlockSpec((FE, H), lambda i: (0, 0)),
            pl.BlockSpec((H,), lambda i: (0,)),
        ],
        out_specs=[
            pl.BlockSpec((blk, 1), lambda i: (i, 0)),
            pl.BlockSpec((blk, 1), lambda i: (i, 0)),
        ],
        out_shape=[
            jax.ShapeDtypeStruct((E, 1), _f32),
            jax.ShapeDtypeStruct((E, 1), _f32),
        ],
    )(edge_attr, we1, a_e1, we2, a_e2)


def _mid_proj_body(numlo_ref, numhi_ref, den_ref, b_ref, wfull_ref,
                   asrc_ref, adst_ref, whalf_ref, h_ref, ss_ref, sd_ref):
    num = jnp.concatenate([numlo_ref[...], numhi_ref[...]], axis=1)
    den = den_ref[...][0, 0][:, None]
    x = jnp.maximum(num / (den + 1e-16) + b_ref[...], 0.0)
    h_ref[...] = jnp.dot(x, whalf_ref[...], preferred_element_type=_f32)

    @pl.when(pl.program_id(1) == 0)
    def _():
        wv_s = jnp.dot(wfull_ref[...], asrc_ref[...],
                       preferred_element_type=_f32)
        wv_d = jnp.dot(wfull_ref[...], adst_ref[...],
                       preferred_element_type=_f32)
        ss_ref[...] = jnp.dot(x, wv_s, preferred_element_type=_f32)[:, None]
        sd_ref[...] = jnp.dot(x, wv_d, preferred_element_type=_f32)[:, None]


def _mid_proj(num, den, b, w, a_src, a_dst):
    nb = 10
    blk = N // nb
    return pl.pallas_call(
        _mid_proj_body,
        grid=(nb, NCORE),
        in_specs=[
            pl.BlockSpec((blk, HH), lambda i, j: (i, 0)),
            pl.BlockSpec((blk, HH), lambda i, j: (nb + i, 0)),
            pl.BlockSpec((1, 1, blk), lambda i, j: (i, 0, 0)),
            pl.BlockSpec((H,), lambda i, j: (0,)),
            pl.BlockSpec((H, H), lambda i, j: (0, 0)),
            pl.BlockSpec((H,), lambda i, j: (0,)),
            pl.BlockSpec((H,), lambda i, j: (0,)),
            pl.BlockSpec((H, HH), lambda i, j: (0, j)),
        ],
        out_specs=[
            pl.BlockSpec((blk, HH), lambda i, j: (j * nb + i, 0)),
            pl.BlockSpec((blk, 1), lambda i, j: (i, 0)),
            pl.BlockSpec((blk, 1), lambda i, j: (i, 0)),
        ],
        out_shape=[
            jax.ShapeDtypeStruct((2 * N, HH), _f32),
            jax.ShapeDtypeStruct((N, 1), _f32),
            jax.ShapeDtypeStruct((N, 1), _f32),
        ],
    )(num, num, den.reshape(nb, 1, blk), b, w, a_src, a_dst, w)


NSTEPS = -(-NCHUNKS // NSUB) * NSUB // NSUB
NPAIRS = (NSTEPS + 1) // 2


def _sc_message_body(h_hbm, ssrc_hbm, sdst_hbm, ae_hbm, src_hbm, dst_hbm,
                     num_hbm, den_hbm,
                     ssrc_v, sdst_v,
                     srcA, dstA, aeA, exA, rowsA,
                     srcB, dstB, aeB, exB, rowsB,
                     ssem, gsem, wsem, acc_sh, den_sh):
    c = lax.axis_index("c")
    s = lax.axis_index("s")

    pltpu.sync_copy(ssrc_hbm, ssrc_v)
    pltpu.sync_copy(sdst_hbm, sdst_v)

    zeros = jnp.zeros((LN,), _f32)

    def _zero_rows(i, _):
        for k in range(HH // LN):
            rowsA[i, pl.ds(k * LN, LN)] = zeros
        return 0
    lax.fori_loop(0, LN, _zero_rows, 0)

    nzc = -(-(N // LN) // NSUB)

    def _zero_acc(i, _):
        ci = jnp.minimum(i * NSUB + s, N // LN - 1)
        pltpu.async_copy(rowsA.at[pl.ds(0, LN)],
                         acc_sh.at[pl.ds(ci * LN, LN)], ssem)
        pltpu.async_copy(rowsA.at[0].at[pl.ds(0, LN)],
                         den_sh.at[pl.ds(ci * LN, LN)], gsem)
        return 0
    lax.fori_loop(0, nzc, _zero_acc, 0)

    def _zero_wait(i, _):
        pltpu.make_async_copy(rowsA.at[pl.ds(0, LN)],
                              acc_sh.at[pl.ds(0, LN)], ssem).wait()
        pltpu.make_async_copy(rowsA.at[0].at[pl.ds(0, LN)],
                              den_sh.at[pl.ds(0, LN)], gsem).wait()
        return 0
    lax.fori_loop(0, nzc, _zero_wait, 0)

    plsc.subcore_barrier()

    coff = c * N


    def _stage(ci, srcv, dstv, aev):
        base = jnp.minimum(ci, NCHUNKS - 1) * CK
        return (pltpu.async_copy(src_hbm.at[pl.ds(base, CK)], srcv, ssem),
                pltpu.async_copy(dst_hbm.at[pl.ds(base, CK)], dstv, ssem),
                pltpu.async_copy(ae_hbm.at[pl.ds(base, CK)], aev, ssem))

    def _wait_w(rowsv, dstv, exv):
        pltpu.make_async_copy(rowsv, acc_sh.at[dstv], wsem).wait()
        pltpu.make_async_copy(exv, den_sh.at[dstv], wsem).wait()

    def _fix_gather(srcv, dstv, rowsv):
        raws, dsts = [], []
        for j in range(CK // LN):
            sl = pl.ds(j * LN, LN)
            sv = srcv[sl]
            raws.append(sv)
            dsts.append(dstv[sl])
            srcv[sl] = sv + coff
        return raws, dsts, pltpu.async_copy(h_hbm.at[srcv], rowsv, gsem)

    def _ex_compute(ci, raws, dsts, aev, exv):
        vf = jnp.where(ci < NCHUNKS, 1.0, 0.0).astype(_f32)
        for j in range(CK // LN):
            sl = pl.ds(j * LN, LN)
            a = (plsc.load_gather(ssrc_v, [raws[j]])
                 + plsc.load_gather(sdst_v, [dsts[j]])
                 + aev[sl])
            a = jnp.maximum(a, 0.2 * a)
            exv[sl] = jnp.exp(a) * vf

    def _scale(rowsv, exv):
        def body(g, _):
            exg = exv[pl.ds(g * LN, LN)]
            for l in range(LN):
                e = exg[l]
                jj = g * LN + l
                for k in range(HH // LN):
                    slk = pl.ds(k * LN, LN)
                    rowsv[jj, slk] = rowsv[jj, slk] * e
            return 0
        lax.fori_loop(0, CK // LN, body, 0)

    def _scatter(rowsv, dstv, exv):
        pltpu.async_copy(rowsv, acc_sh.at[dstv], wsem, add=True)
        pltpu.async_copy(exv, den_sh.at[dstv], wsem, add=True)

    def _pair(i, _):
        ci0 = (2 * i) * NSUB + s
        ci1 = (2 * i + 1) * NSUB + s

        @pl.when(i > 0)
        def _():
            _wait_w(rowsA, dstA, exA)
            _wait_w(rowsB, dstB, exB)

        stA = _stage(ci0, srcA, dstA, aeA)
        stB = _stage(ci1, srcB, dstB, aeB)
        for d in stA:
            d.wait()
        rawsA, dstsA, gA = _fix_gather(srcA, dstA, rowsA)
        for d in stB:
            d.wait()
        rawsB, dstsB, gB = _fix_gather(srcB, dstB, rowsB)

        _ex_compute(ci0, rawsA, dstsA, aeA, exA)
        _ex_compute(ci1, rawsB, dstsB, aeB, exB)

        gA.wait()
        _scale(rowsA, exA)
        _scatter(rowsA, dstA, exA)

        gB.wait()
        _scale(rowsB, exB)
        _scatter(rowsB, dstB, exB)
        return 0

    lax.fori_loop(0, NPAIRS, _pair, 0)

    _wait_w(rowsA, dstA, exA)
    _wait_w(rowsB, dstB, exB)

    plsc.subcore_barrier()

    def _out_copy(i, _):
        ci = jnp.minimum(i * NSUB + s, N // LN - 1)
        pltpu.async_copy(acc_sh.at[pl.ds(ci * LN, LN)],
                         num_hbm.at[pl.ds(coff + ci * LN, LN)], ssem)
        return 0
    lax.fori_loop(0, nzc, _out_copy, 0)

    def _out_wait(i, _):
        pltpu.make_async_copy(acc_sh.at[pl.ds(0, LN)],
                              num_hbm.at[pl.ds(coff, LN)], ssem).wait()
        return 0
    lax.fori_loop(0, nzc, _out_wait, 0)

    @pl.when((s == 0) & (c == 0))
    def _():
        pltpu.sync_copy(den_sh, den_hbm)


_sc_message = functools.partial(
    pl.kernel,
    _sc_message_body,
    out_type=(
        jax.ShapeDtypeStruct((2 * N, HH), _f32),
        jax.ShapeDtypeStruct((N,), _f32),
    ),
    mesh=plsc.VectorSubcoreMesh(core_axis_name="c", subcore_axis_name="s",
                                num_cores=NCORE, num_subcores=NSUB),
    compiler_params=pltpu.CompilerParams(needs_layout_passes=False),
    scratch_types=[
        pltpu.VMEM((N,), _f32),
        pltpu.VMEM((N,), _f32),
        pltpu.VMEM((CK,), jnp.int32),
        pltpu.VMEM((CK,), jnp.int32),
        pltpu.VMEM((CK,), _f32),
        pltpu.VMEM((CK,), _f32),
        pltpu.VMEM((CK, HH), _f32),
        pltpu.VMEM((CK,), jnp.int32),
        pltpu.VMEM((CK,), jnp.int32),
        pltpu.VMEM((CK,), _f32),
        pltpu.VMEM((CK,), _f32),
        pltpu.VMEM((CK, HH), _f32),
        pltpu.SemaphoreType.DMA,
        pltpu.SemaphoreType.DMA,
        pltpu.SemaphoreType.DMA,
        pltpu.VMEM_SHARED((N, HH), _f32),
        pltpu.VMEM_SHARED((N,), _f32),
    ],
)()


NPT = 632
NPT_LAST = N - (NSUB - 1) * NPT


def _sc_pool_body(num_hbm, den_hbm, b2_hbm, batch_hbm,
                  gsum_hbm, gmax_hbm,
                  x_v, den_v, batch_v, b2_v, sum_v, max_v, mg_v, pool_sh):
    c = lax.axis_index("c")
    s = lax.axis_index("s")
    base = s * NPT
    is_last = s == NSUB - 1

    @pl.when(is_last)
    def _():
        pltpu.sync_copy(num_hbm.at[pl.ds(c * N + base, NPT_LAST)],
                        x_v.at[pl.ds(0, NPT_LAST)])
        pltpu.sync_copy(den_hbm.at[pl.ds(base, NPT_LAST)],
                        den_v.at[pl.ds(0, NPT_LAST)])
        pltpu.sync_copy(batch_hbm.at[pl.ds(base, NPT_LAST)],
                        batch_v.at[pl.ds(0, NPT_LAST)])

    @pl.when(jnp.logical_not(is_last))
    def _():
        pltpu.sync_copy(num_hbm.at[pl.ds(c * N + base, NPT)], x_v)
        pltpu.sync_copy(den_hbm.at[pl.ds(base, NPT)], den_v)
        pltpu.sync_copy(batch_hbm.at[pl.ds(base, NPT)], batch_v)

    pltpu.sync_copy(b2_hbm, b2_v)

    zeros = jnp.zeros((LN,), _f32)

    def _zero_acc(i, _):
        for k in range(HH // LN):
            sum_v[i, pl.ds(k * LN, LN)] = zeros
            max_v[i, pl.ds(k * LN, LN)] = zeros
        return 0
    lax.fori_loop(0, B, _zero_acc, 0)

    b2c = [b2_v[pl.ds(c * HH + k * LN, LN)] for k in range(HH // LN)]

    sz = jnp.where(is_last, NPT_LAST, NPT)
    ngroups = jnp.where(is_last, NPT_LAST // LN, NPT // LN)

    def _do_node(n, b, rec):
        for k in range(HH // LN):
            slk = pl.ds(k * LN, LN)
            xa = jnp.maximum(x_v[n, slk] * rec + b2c[k], 0.0)
            sum_v[b, slk] = sum_v[b, slk] + xa
            max_v[b, slk] = jnp.maximum(max_v[b, slk], xa)

    def _node(g, _):
        @pl.when(g < ngroups)
        def _():
            bv = batch_v[pl.ds(g * LN, LN)]
            dv = den_v[pl.ds(g * LN, LN)]
            recv = 1.0 / (dv + 1e-16)
            for l in range(LN):
                _do_node(g * LN + l, bv[l], recv[l])
        return 0
    lax.fori_loop(0, NPT // LN, _node, 0)

    bv = batch_v[pl.ds(sz - LN, LN)]
    dv = den_v[pl.ds(sz - LN, LN)]
    recv = 1.0 / (dv + 1e-16)
    for l in range(LN // 2, LN):
        _do_node(sz - LN + l, bv[l], recv[l])

    for acc_v, out_hbm, is_max in ((sum_v, gsum_hbm, False),
                                   (max_v, gmax_hbm, True)):
        pltpu.sync_copy(acc_v, pool_sh.at[s])
        plsc.subcore_barrier()

        @pl.when(s < NSUB // 2)
        def _():
            for t in range(NSUB):
                pltpu.sync_copy(pool_sh.at[t].at[pl.ds(s * SEG_PT, SEG_PT)],
                                mg_v.at[pl.ds(t * SEG_PT, SEG_PT)])

            def _merge_row(r, _):
                for k in range(HH // LN):
                    slk = pl.ds(k * LN, LN)
                    v = mg_v[r, slk]
                    for t in range(1, NSUB):
                        if is_max:
                            v = jnp.maximum(v, mg_v[t * SEG_PT + r, slk])
                        else:
                            v = v + mg_v[t * SEG_PT + r, slk]
                    sum_v[r, slk] = v
                return 0
            lax.fori_loop(0, SEG_PT, _merge_row, 0)
            pltpu.sync_copy(sum_v.at[pl.ds(0, SEG_PT)],
                            out_hbm.at[pl.ds(c * B + s * SEG_PT, SEG_PT)])
        plsc.subcore_barrier()


_sc_pool = functools.partial(
    pl.kernel,
    _sc_pool_body,
    out_type=(
        jax.ShapeDtypeStruct((NCORE * B, HH), _f32),
        jax.ShapeDtypeStruct((NCORE * B, HH), _f32),
    ),
    mesh=plsc.VectorSubcoreMesh(core_axis_name="c", subcore_axis_name="s",
                                num_cores=NCORE, num_subcores=NSUB),
    compiler_params=pltpu.CompilerParams(needs_layout_passes=False),
    scratch_types=[
        pltpu.VMEM((NPT, HH), _f32),
        pltpu.VMEM((NPT,), _f32),
        pltpu.VMEM((NPT,), jnp.int32),
        pltpu.VMEM((H,), _f32),
        pltpu.VMEM((B, HH), _f32),
        pltpu.VMEM((B, HH), _f32),
        pltpu.VMEM((NSUB * SEG_PT, HH), _f32),
        pltpu.VMEM_SHARED((NSUB, B, HH), _f32),
    ],
)()


def _final_body(gsum_ref, gmax_ref, batch_ref, clin_ref, wc1_ref, bc1_ref,
                wc2_ref, bc2_ref, wg1_ref, bg1_ref, wg2_ref, bg2_ref,
                wcls_ref, bcls_ref, o_ref):
    bt = batch_ref[...]
    eq = bt[:, None] == lax.broadcasted_iota(jnp.int32, (N, B), 1)
    cnt = jnp.sum(jnp.where(eq, 1.0, 0.0), axis=0)
    cnt = jnp.clip(cnt, 1.0, None)

    gsum = jnp.concatenate([gsum_ref[...][:B], gsum_ref[...][B:]], axis=1)
    gmax = jnp.concatenate([gmax_ref[...][:B], gmax_ref[...][B:]], axis=1)
    gmean = gsum / cnt[:, None]
    emb_g = jnp.concatenate([gmean, gmax], axis=1)

    hc = jnp.maximum(
        jnp.dot(clin_ref[...], wc1_ref[...], preferred_element_type=_f32)
        + bc1_ref[...], 0.0)
    emb_c = (jnp.dot(hc, wc2_ref[...], preferred_element_type=_f32)
             + bc2_ref[...])

    ng = jnp.sqrt(jnp.sum(emb_g * emb_g, axis=1, keepdims=True))
    emb_g = emb_g / jnp.maximum(ng, 1e-12)
    nc = jnp.sqrt(jnp.sum(emb_c * emb_c, axis=1, keepdims=True))
    emb_c = emb_c / jnp.maximum(nc, 1e-12)

    cat = jnp.concatenate([emb_g, emb_c], axis=1)
    g1 = jnp.maximum(
        jnp.dot(cat, wg1_ref[...], preferred_element_type=_f32)
        + bg1_ref[...], 0.0)
    gate = jax.nn.sigmoid(
        jnp.dot(g1, wg2_ref[...], preferred_element_type=_f32) + bg2_ref[...])
    comb = jnp.concatenate([emb_g, emb_c * gate], axis=-1)
    o_ref[...] = (jnp.dot(comb, wcls_ref[...], preferred_element_type=_f32)
                  + bcls_ref[...])


def _final(gsum, gmax, batch, clinical, wc1, bc1, wc2, bc2,
           wg1, bg1, wg2, bg2, wcls, bcls):
    return pl.pallas_call(
        _final_body,
        out_shape=jax.ShapeDtypeStruct((B, NCLS), _f32),
    )(gsum, gmax, batch, clinical, wc1, bc1, wc2, bc2,
      wg1, bg1, wg2, bg2, wcls, bcls)


def kernel(graph_data, edge_index, edge_attr, clinical_data, batch,
           W1, a_src1, a_dst1, We1, a_e1, b1,
           W2, a_src2, a_dst2, We2, a_e2, b2,
           Wc1, bc1, Wc2, bc2, Wg1, bg1, Wg2, bg2, Wcls, bcls):
    src = edge_index[0]
    dst = edge_index[1]

    h1, s1s, s1d = _node_proj(graph_data, W1, a_src1, a_dst1)
    ae1, ae2 = _edge_proj(edge_attr, We1, a_e1, We2, a_e2)

    num1, den1 = _sc_message(h1, jnp.squeeze(s1s, 1), jnp.squeeze(s1d, 1),
                             jnp.squeeze(ae1, 1), src, dst)

    h2, s2s, s2d = _mid_proj(num1, den1, b1, W2, a_src2, a_dst2)

    num2, den2 = _sc_message(h2, jnp.squeeze(s2s, 1), jnp.squeeze(s2d, 1),
                             jnp.squeeze(ae2, 1), src, dst)

    gsum, gmax = _sc_pool(num2, den2, b2, batch)

    return _final(gsum, gmax, batch, clinical_data,
                  Wc1, bc1, Wc2, bc2, Wg1, bg1, Wg2, bg2, Wcls, bcls)


# speedup vs baseline: 13.6404x; 1.0452x over previous
import functools

import jax
import jax.numpy as jnp
from jax import lax
from jax.experimental import pallas as pl
from jax.experimental.pallas import tpu as pltpu
from jax.experimental.pallas import tpu_sc as plsc

N = 10000
E = 160000
F = 256
FE = 16
H = 256
B = 64
CLIN = 32
NCLS = 4

HH = H // 2
NCORE = 2
NSUB = 16
LN = 16
CK = 80
NCHUNKS = E // CK
ITERS = -(-NCHUNKS // NSUB)
RPT = N // NSUB
SEG_PT = B // (NSUB // 2)

_f32 = jnp.float32


def _node_proj_body(x_ref, wfull_ref, asrc_ref, adst_ref, whalf_ref,
                    h_ref, ss_ref, sd_ref):
    x = x_ref[...]
    h_ref[...] = jnp.dot(x, whalf_ref[...], preferred_element_type=_f32)

    @pl.when(pl.program_id(1) == 0)
    def _():
        wv_s = jnp.dot(wfull_ref[...], asrc_ref[...],
                       preferred_element_type=_f32)
        wv_d = jnp.dot(wfull_ref[...], adst_ref[...],
                       preferred_element_type=_f32)
        ss_ref[...] = jnp.dot(x, wv_s, preferred_element_type=_f32)[:, None]
        sd_ref[...] = jnp.dot(x, wv_d, preferred_element_type=_f32)[:, None]


def _node_proj(x, w, a_src, a_dst):
    nb = 10
    blk = N // nb
    return pl.pallas_call(
        _node_proj_body,
        grid=(nb, NCORE),
        in_specs=[
            pl.BlockSpec((blk, F), lambda i, j: (i, 0)),
            pl.BlockSpec((F, H), lambda i, j: (0, 0)),
            pl.BlockSpec((H,), lambda i, j: (0,)),
            pl.BlockSpec((H,), lambda i, j: (0,)),
            pl.BlockSpec((F, HH), lambda i, j: (0, j)),
        ],
        out_specs=[
            pl.BlockSpec((blk, HH), lambda i, j: (j * nb + i, 0)),
            pl.BlockSpec((blk, 1), lambda i, j: (i, 0)),
            pl.BlockSpec((blk, 1), lambda i, j: (i, 0)),
        ],
        out_shape=[
            jax.ShapeDtypeStruct((2 * N, HH), _f32),
            jax.ShapeDtypeStruct((N, 1), _f32),
            jax.ShapeDtypeStruct((N, 1), _f32),
        ],
    )(x, w, a_src, a_dst, w)


def _edge_proj_body(ea_ref, we1_ref, ae1v_ref, we2_ref, ae2v_ref,
                    o1_ref, o2_ref):
    ea = ea_ref[...]
    w1 = jnp.dot(we1_ref[...], ae1v_ref[...], preferred_element_type=_f32)
    w2 = jnp.dot(we2_ref[...], ae2v_ref[...], preferred_element_type=_f32)
    o1_ref[...] = jnp.dot(ea, w1, preferred_element_type=_f32)[:, None]
    o2_ref[...] = jnp.dot(ea, w2, preferred_element_type=_f32)[:, None]


def _edge_proj(edge_attr, we1, a_e1, we2, a_e2):
    nb = 80
    blk = E // nb
    return pl.pallas_call(
        _edge_proj_body,
        grid=(nb,),
        in_specs=[
            pl.BlockSpec((blk, FE), lambda i: (i, 0)),
            pl.BlockSpec((FE, H), lambda i: (0, 0)),
            pl.BlockSpec((H,), lambda i: (0,)),
            pl.BlockSpec((FE, H), lambda i: (0, 0)),
            pl.BlockSpec((H,), lambda i: (0,)),
        ],
        out_specs=[
            pl.BlockSpec((blk, 1), lambda i: (i, 0)),
            pl.BlockSpec((blk, 1), lambda i: (i, 0)),
        ],
        out_shape=[
            jax.ShapeDtypeStruct((E, 1), _f32),
            jax.ShapeDtypeStruct((E, 1), _f32),
        ],
    )(edge_attr, we1, a_e1, we2, a_e2)


def _mid_proj_body(numlo_ref, numhi_ref, den_ref, b_ref, wfull_ref,
                   asrc_ref, adst_ref, whalf_ref, h_ref, ss_ref, sd_ref):
    num = jnp.concatenate([numlo_ref[...], numhi_ref[...]], axis=1)
    den = den_ref[...][0, 0][:, None]
    x = jnp.maximum(num / (den + 1e-16) + b_ref[...], 0.0)
    h_ref[...] = jnp.dot(x, whalf_ref[...], preferred_element_type=_f32)

    @pl.when(pl.program_id(1) == 0)
    def _():
        wv_s = jnp.dot(wfull_ref[...], asrc_ref[...],
                       preferred_element_type=_f32)
        wv_d = jnp.dot(wfull_ref[...], adst_ref[...],
                       preferred_element_type=_f32)
        ss_ref[...] = jnp.dot(x, wv_s, preferred_element_type=_f32)[:, None]
        sd_ref[...] = jnp.dot(x, wv_d, preferred_element_type=_f32)[:, None]


def _mid_proj(num, den, b, w, a_src, a_dst):
    nb = 10
    blk = N // nb
    return pl.pallas_call(
        _mid_proj_body,
        grid=(nb, NCORE),
        in_specs=[
            pl.BlockSpec((blk, HH), lambda i, j: (i, 0)),
            pl.BlockSpec((blk, HH), lambda i, j: (nb + i, 0)),
            pl.BlockSpec((1, 1, blk), lambda i, j: (i, 0, 0)),
            pl.BlockSpec((H,), lambda i, j: (0,)),
            pl.BlockSpec((H, H), lambda i, j: (0, 0)),
            pl.BlockSpec((H,), lambda i, j: (0,)),
            pl.BlockSpec((H,), lambda i, j: (0,)),
            pl.BlockSpec((H, HH), lambda i, j: (0, j)),
        ],
        out_specs=[
            pl.BlockSpec((blk, HH), lambda i, j: (j * nb + i, 0)),
            pl.BlockSpec((blk, 1), lambda i, j: (i, 0)),
            pl.BlockSpec((blk, 1), lambda i, j: (i, 0)),
        ],
        out_shape=[
            jax.ShapeDtypeStruct((2 * N, HH), _f32),
            jax.ShapeDtypeStruct((N, 1), _f32),
            jax.ShapeDtypeStruct((N, 1), _f32),
        ],
    )(num, num, den.reshape(nb, 1, blk), b, w, a_src, a_dst, w)


NSTEPS = -(-NCHUNKS // NSUB)
NPAIRS = (NSTEPS + 1) // 2
NSUPER = NPAIRS // 2
assert NPAIRS % 2 == 1


def _sc_message_body(h_hbm, ssrc_hbm, sdst_hbm, ae_hbm, src_hbm, dst_hbm,
                     num_hbm, den_hbm,
                     ssrc_v, sdst_v,
                     src0, dst0, ae0, ex0, src1, dst1, ae1, ex1,
                     src2, dst2, ae2, ex2, src3, dst3, ae3, ex3,
                     rowsA, rowsB,
                     zsem, gsem, semA, semB, wsem, acc_sh, den_sh):
    S01 = ((src0, dst0, ae0, ex0), (src1, dst1, ae1, ex1))
    S23 = ((src2, dst2, ae2, ex2), (src3, dst3, ae3, ex3))
    c = lax.axis_index("c")
    s = lax.axis_index("s")

    pltpu.sync_copy(ssrc_hbm, ssrc_v)
    pltpu.sync_copy(sdst_hbm, sdst_v)

    zeros = jnp.zeros((LN,), _f32)

    def _zero_rows(i, _):
        for k in range(HH // LN):
            rowsA[i, pl.ds(k * LN, LN)] = zeros
        return 0
    lax.fori_loop(0, LN, _zero_rows, 0)

    nzc = -(-(N // LN) // NSUB)

    def _zero_acc(i, _):
        ci = jnp.minimum(i * NSUB + s, N // LN - 1)
        pltpu.async_copy(rowsA.at[pl.ds(0, LN)],
                         acc_sh.at[pl.ds(ci * LN, LN)], zsem)
        pltpu.async_copy(rowsA.at[0].at[pl.ds(0, LN)],
                         den_sh.at[pl.ds(ci * LN, LN)], gsem)
        return 0
    lax.fori_loop(0, nzc, _zero_acc, 0)

    coff = c * N


    def _stage_pair(pi, SX, sem):
        for k in range(2):
            ci = (2 * pi + k) * NSUB + s
            base = jnp.minimum(ci, NCHUNKS - 1) * CK
            srcv, dstv, aev, _ = SX[k]
            pltpu.async_copy(src_hbm.at[pl.ds(base, CK)], srcv, sem)
            pltpu.async_copy(dst_hbm.at[pl.ds(base, CK)], dstv, sem)
            pltpu.async_copy(ae_hbm.at[pl.ds(base, CK)], aev, sem)

    def _wait_stage_pair(SX, sem):
        for k in range(2):
            srcv, dstv, aev, _ = SX[k]
            pltpu.make_async_copy(src_hbm.at[pl.ds(0, CK)], srcv, sem).wait()
            pltpu.make_async_copy(dst_hbm.at[pl.ds(0, CK)], dstv, sem).wait()
            pltpu.make_async_copy(ae_hbm.at[pl.ds(0, CK)], aev, sem).wait()

    _stage_pair(0, S01, semA)

    def _zero_wait(i, _):
        pltpu.make_async_copy(rowsA.at[pl.ds(0, LN)],
                              acc_sh.at[pl.ds(0, LN)], zsem).wait()
        pltpu.make_async_copy(rowsA.at[0].at[pl.ds(0, LN)],
                              den_sh.at[pl.ds(0, LN)], gsem).wait()
        return 0
    lax.fori_loop(0, nzc, _zero_wait, 0)

    plsc.subcore_barrier()

    def _wait_w(rowsv, dstv, exv):
        pltpu.make_async_copy(rowsv, acc_sh.at[dstv], wsem).wait()
        pltpu.make_async_copy(exv, den_sh.at[dstv], wsem).wait()

    def _fix_gather(srcv, dstv, rowsv):
        raws, dsts = [], []
        for j in range(CK // LN):
            sl = pl.ds(j * LN, LN)
            sv = srcv[sl]
            raws.append(sv)
            dsts.append(dstv[sl])
            srcv[sl] = sv + coff
        return raws, dsts, pltpu.async_copy(h_hbm.at[srcv], rowsv, gsem)

    def _ex_compute(ci, raws, dsts, aev, exv):
        vf = jnp.where(ci < NCHUNKS, 1.0, 0.0).astype(_f32)
        for j in range(CK // LN):
            sl = pl.ds(j * LN, LN)
            a = (plsc.load_gather(ssrc_v, [raws[j]])
                 + plsc.load_gather(sdst_v, [dsts[j]])
                 + aev[sl])
            a = jnp.maximum(a, 0.2 * a)
            exv[sl] = jnp.exp(a) * vf

    def _scale(rowsv, exv):
        def body(g, _):
            exg = exv[pl.ds(g * LN, LN)]
            for l in range(LN):
                e = exg[l]
                jj = g * LN + l
                for k in range(HH // LN):
                    slk = pl.ds(k * LN, LN)
                    rowsv[jj, slk] = rowsv[jj, slk] * e
            return 0
        lax.fori_loop(0, CK // LN, body, 0)

    def _scatter(rowsv, dstv, exv):
        pltpu.async_copy(rowsv, acc_sh.at[dstv], wsem, add=True)
        pltpu.async_copy(exv, den_sh.at[dstv], wsem, add=True)

    def _proc_pair(pi, SX):
        ci0 = (2 * pi) * NSUB + s
        ci1 = (2 * pi + 1) * NSUB + s
        raws0, dsts0, g0 = _fix_gather(SX[0][0], SX[0][1], rowsA)
        raws1, dsts1, g1 = _fix_gather(SX[1][0], SX[1][1], rowsB)
        _ex_compute(ci0, raws0, dsts0, SX[0][2], SX[0][3])
        _ex_compute(ci1, raws1, dsts1, SX[1][2], SX[1][3])
        g0.wait()
        _scale(rowsA, SX[0][3])
        _scatter(rowsA, SX[0][1], SX[0][3])
        g1.wait()
        _scale(rowsB, SX[1][3])
        _scatter(rowsB, SX[1][1], SX[1][3])

    def _wait_pair(SX):
        _wait_w(rowsA, SX[0][1], SX[0][3])
        _wait_w(rowsB, SX[1][1], SX[1][3])

    def _super(k, _):
        @pl.when(k > 0)
        def _():
            _wait_pair(S23)
        _stage_pair(2 * k + 1, S23, semB)
        _wait_stage_pair(S01, semA)
        _proc_pair(2 * k, S01)

        _wait_pair(S01)
        _stage_pair(2 * k + 2, S01, semA)
        _wait_stage_pair(S23, semB)
        _proc_pair(2 * k + 1, S23)
        return 0

    lax.fori_loop(0, NSUPER, _super, 0)

    _wait_pair(S23)
    _wait_stage_pair(S01, semA)
    _proc_pair(NPAIRS - 1, S01)
    _wait_pair(S01)

    plsc.subcore_barrier()

    def _out_copy(i, _):
        ci = jnp.minimum(i * NSUB + s, N // LN - 1)
        pltpu.async_copy(acc_sh.at[pl.ds(ci * LN, LN)],
                         num_hbm.at[pl.ds(coff + ci * LN, LN)], zsem)
        return 0
    lax.fori_loop(0, nzc, _out_copy, 0)

    def _out_wait(i, _):
        pltpu.make_async_copy(acc_sh.at[pl.ds(0, LN)],
                              num_hbm.at[pl.ds(coff, LN)], zsem).wait()
        return 0
    lax.fori_loop(0, nzc, _out_wait, 0)

    @pl.when((s == 0) & (c == 0))
    def _():
        pltpu.sync_copy(den_sh, den_hbm)


_sc_message = functools.partial(
    pl.kernel,
    _sc_message_body,
    out_type=(
        jax.ShapeDtypeStruct((2 * N, HH), _f32),
        jax.ShapeDtypeStruct((N,), _f32),
    ),
    mesh=plsc.VectorSubcoreMesh(core_axis_name="c", subcore_axis_name="s",
                                num_cores=NCORE, num_subcores=NSUB),
    compiler_params=pltpu.CompilerParams(needs_layout_passes=False),
    scratch_types=[
        pltpu.VMEM((N,), _f32),
        pltpu.VMEM((N,), _f32),
        pltpu.VMEM((CK,), jnp.int32),
        pltpu.VMEM((CK,), jnp.int32),
        pltpu.VMEM((CK,), _f32),
        pltpu.VMEM((CK,), _f32),
        pltpu.VMEM((CK,), jnp.int32),
        pltpu.VMEM((CK,), jnp.int32),
        pltpu.VMEM((CK,), _f32),
        pltpu.VMEM((CK,), _f32),
        pltpu.VMEM((CK,), jnp.int32),
        pltpu.VMEM((CK,), jnp.int32),
        pltpu.VMEM((CK,), _f32),
        pltpu.VMEM((CK,), _f32),
        pltpu.VMEM((CK,), jnp.int32),
        pltpu.VMEM((CK,), jnp.int32),
        pltpu.VMEM((CK,), _f32),
        pltpu.VMEM((CK,), _f32),
        pltpu.VMEM((CK, HH), _f32),
        pltpu.VMEM((CK, HH), _f32),
        pltpu.SemaphoreType.DMA,
        pltpu.SemaphoreType.DMA,
        pltpu.SemaphoreType.DMA,
        pltpu.SemaphoreType.DMA,
        pltpu.SemaphoreType.DMA,
        pltpu.VMEM_SHARED((N, HH), _f32),
        pltpu.VMEM_SHARED((N,), _f32),
    ],
)()


NPT = 632
NPT_LAST = N - (NSUB - 1) * NPT


def _sc_pool_body(num_hbm, den_hbm, b2_hbm, batch_hbm,
                  gsum_hbm, gmax_hbm,
                  x_v, den_v, batch_v, b2_v, sum_v, max_v, mg_v, pool_sh):
    c = lax.axis_index("c")
    s = lax.axis_index("s")
    base = s * NPT
    is_last = s == NSUB - 1

    @pl.when(is_last)
    def _():
        pltpu.sync_copy(num_hbm.at[pl.ds(c * N + base, NPT_LAST)],
                        x_v.at[pl.ds(0, NPT_LAST)])
        pltpu.sync_copy(den_hbm.at[pl.ds(base, NPT_LAST)],
                        den_v.at[pl.ds(0, NPT_LAST)])
        pltpu.sync_copy(batch_hbm.at[pl.ds(base, NPT_LAST)],
                        batch_v.at[pl.ds(0, NPT_LAST)])

    @pl.when(jnp.logical_not(is_last))
    def _():
        pltpu.sync_copy(num_hbm.at[pl.ds(c * N + base, NPT)], x_v)
        pltpu.sync_copy(den_hbm.at[pl.ds(base, NPT)], den_v)
        pltpu.sync_copy(batch_hbm.at[pl.ds(base, NPT)], batch_v)

    pltpu.sync_copy(b2_hbm, b2_v)

    zeros = jnp.zeros((LN,), _f32)

    def _zero_acc(i, _):
        for k in range(HH // LN):
            sum_v[i, pl.ds(k * LN, LN)] = zeros
            max_v[i, pl.ds(k * LN, LN)] = zeros
        return 0
    lax.fori_loop(0, B, _zero_acc, 0)

    b2c = [b2_v[pl.ds(c * HH + k * LN, LN)] for k in range(HH // LN)]

    sz = jnp.where(is_last, NPT_LAST, NPT)
    ngroups = jnp.where(is_last, NPT_LAST // LN, NPT // LN)

    def _do_node(n, b, rec):
        for k in range(HH // LN):
            slk = pl.ds(k * LN, LN)
            xa = jnp.maximum(x_v[n, slk] * rec + b2c[k], 0.0)
            sum_v[b, slk] = sum_v[b, slk] + xa
            max_v[b, slk] = jnp.maximum(max_v[b, slk], xa)

    def _node(g, _):
        @pl.when(g < ngroups)
        def _():
            bv = batch_v[pl.ds(g * LN, LN)]
            dv = den_v[pl.ds(g * LN, LN)]
            recv = 1.0 / (dv + 1e-16)
            for l in range(LN):
                _do_node(g * LN + l, bv[l], recv[l])
        return 0
    lax.fori_loop(0, NPT // LN, _node, 0)

    bv = batch_v[pl.ds(sz - LN, LN)]
    dv = den_v[pl.ds(sz - LN, LN)]
    recv = 1.0 / (dv + 1e-16)
    for l in range(LN // 2, LN):
        _do_node(sz - LN + l, bv[l], recv[l])

    for acc_v, out_hbm, is_max in ((sum_v, gsum_hbm, False),
                                   (max_v, gmax_hbm, True)):
        pltpu.sync_copy(acc_v, pool_sh.at[s])
        plsc.subcore_barrier()

        @pl.when(s < NSUB // 2)
        def _():
            for t in range(NSUB):
                pltpu.sync_copy(pool_sh.at[t].at[pl.ds(s * SEG_PT, SEG_PT)],
                                mg_v.at[pl.ds(t * SEG_PT, SEG_PT)])

            def _merge_row(r, _):
                for k in range(HH // LN):
                    slk = pl.ds(k * LN, LN)
                    v = mg_v[r, slk]
                    for t in range(1, NSUB):
                        if is_max:
                            v = jnp.maximum(v, mg_v[t * SEG_PT + r, slk])
                        else:
                            v = v + mg_v[t * SEG_PT + r, slk]
                    sum_v[r, slk] = v
                return 0
            lax.fori_loop(0, SEG_PT, _merge_row, 0)
            pltpu.sync_copy(sum_v.at[pl.ds(0, SEG_PT)],
                            out_hbm.at[pl.ds(c * B + s * SEG_PT, SEG_PT)])
        plsc.subcore_barrier()


_sc_pool = functools.partial(
    pl.kernel,
    _sc_pool_body,
    out_type=(
        jax.ShapeDtypeStruct((NCORE * B, HH), _f32),
        jax.ShapeDtypeStruct((NCORE * B, HH), _f32),
    ),
    mesh=plsc.VectorSubcoreMesh(core_axis_name="c", subcore_axis_name="s",
                                num_cores=NCORE, num_subcores=NSUB),
    compiler_params=pltpu.CompilerParams(needs_layout_passes=False),
    scratch_types=[
        pltpu.VMEM((NPT, HH), _f32),
        pltpu.VMEM((NPT,), _f32),
        pltpu.VMEM((NPT,), jnp.int32),
        pltpu.VMEM((H,), _f32),
        pltpu.VMEM((B, HH), _f32),
        pltpu.VMEM((B, HH), _f32),
        pltpu.VMEM((NSUB * SEG_PT, HH), _f32),
        pltpu.VMEM_SHARED((NSUB, B, HH), _f32),
    ],
)()


def _final_body(gsum_ref, gmax_ref, batch_ref, clin_ref, wc1_ref, bc1_ref,
                wc2_ref, bc2_ref, wg1_ref, bg1_ref, wg2_ref, bg2_ref,
                wcls_ref, bcls_ref, o_ref):
    bt = batch_ref[...]
    eq = bt[:, None] == lax.broadcasted_iota(jnp.int32, (N, B), 1)
    cnt = jnp.sum(jnp.where(eq, 1.0, 0.0), axis=0)
    cnt = jnp.clip(cnt, 1.0, None)

    gsum = jnp.concatenate([gsum_ref[...][:B], gsum_ref[...][B:]], axis=1)
    gmax = jnp.concatenate([gmax_ref[...][:B], gmax_ref[...][B:]], axis=1)
    gmean = gsum / cnt[:, None]
    emb_g = jnp.concatenate([gmean, gmax], axis=1)

    hc = jnp.maximum(
        jnp.dot(clin_ref[...], wc1_ref[...], preferred_element_type=_f32)
        + bc1_ref[...], 0.0)
    emb_c = (jnp.dot(hc, wc2_ref[...], preferred_element_type=_f32)
             + bc2_ref[...])

    ng = jnp.sqrt(jnp.sum(emb_g * emb_g, axis=1, keepdims=True))
    emb_g = emb_g / jnp.maximum(ng, 1e-12)
    nc = jnp.sqrt(jnp.sum(emb_c * emb_c, axis=1, keepdims=True))
    emb_c = emb_c / jnp.maximum(nc, 1e-12)

    cat = jnp.concatenate([emb_g, emb_c], axis=1)
    g1 = jnp.maximum(
        jnp.dot(cat, wg1_ref[...], preferred_element_type=_f32)
        + bg1_ref[...], 0.0)
    gate = jax.nn.sigmoid(
        jnp.dot(g1, wg2_ref[...], preferred_element_type=_f32) + bg2_ref[...])
    comb = jnp.concatenate([emb_g, emb_c * gate], axis=-1)
    o_ref[...] = (jnp.dot(comb, wcls_ref[...], preferred_element_type=_f32)
                  + bcls_ref[...])


def _final(gsum, gmax, batch, clinical, wc1, bc1, wc2, bc2,
           wg1, bg1, wg2, bg2, wcls, bcls):
    return pl.pallas_call(
        _final_body,
        out_shape=jax.ShapeDtypeStruct((B, NCLS), _f32),
    )(gsum, gmax, batch, clinical, wc1, bc1, wc2, bc2,
      wg1, bg1, wg2, bg2, wcls, bcls)


def kernel(graph_data, edge_index, edge_attr, clinical_data, batch,
           W1, a_src1, a_dst1, We1, a_e1, b1,
           W2, a_src2, a_dst2, We2, a_e2, b2,
           Wc1, bc1, Wc2, bc2, Wg1, bg1, Wg2, bg2, Wcls, bcls):
    src = edge_index[0]
    dst = edge_index[1]

    h1, s1s, s1d = _node_proj(graph_data, W1, a_src1, a_dst1)
    ae1, ae2 = _edge_proj(edge_attr, We1, a_e1, We2, a_e2)

    num1, den1 = _sc_message(h1, jnp.squeeze(s1s, 1), jnp.squeeze(s1d, 1),
                             jnp.squeeze(ae1, 1), src, dst)

    h2, s2s, s2d = _mid_proj(num1, den1, b1, W2, a_src2, a_dst2)

    num2, den2 = _sc_message(h2, jnp.squeeze(s2s, 1), jnp.squeeze(s2d, 1),
                             jnp.squeeze(ae2, 1), src, dst)

    gsum, gmax = _sc_pool(num2, den2, b2, batch)

    return _final(gsum, gmax, batch, clinical_data,
                  Wc1, bc1, Wc2, bc2, Wg1, bg1, Wg2, bg2, Wcls, bcls)


# speedup vs baseline: 14.8072x; 1.0855x over previous
import functools

import jax
import jax.numpy as jnp
from jax import lax
from jax.experimental import pallas as pl
from jax.experimental.pallas import tpu as pltpu
from jax.experimental.pallas import tpu_sc as plsc

N = 10000
E = 160000
F = 256
FE = 16
H = 256
B = 64
CLIN = 32
NCLS = 4

HH = H // 2
NCORE = 2
NSUB = 16
LN = 16
CK = 64
NCHUNKS = E // CK
ITERS = -(-NCHUNKS // NSUB)
RPT = N // NSUB
SEG_PT = B // (NSUB // 2)

_f32 = jnp.float32


def _node_proj_body(x_ref, wfull_ref, asrc_ref, adst_ref, whalf_ref,
                    h_ref, ss_ref, sd_ref):
    x = x_ref[...]
    h_ref[...] = jnp.dot(x, whalf_ref[...], preferred_element_type=_f32)

    @pl.when(pl.program_id(1) == 0)
    def _():
        wv_s = jnp.dot(wfull_ref[...], asrc_ref[...],
                       preferred_element_type=_f32)
        wv_d = jnp.dot(wfull_ref[...], adst_ref[...],
                       preferred_element_type=_f32)
        ss_ref[...] = jnp.dot(x, wv_s, preferred_element_type=_f32)[:, None]
        sd_ref[...] = jnp.dot(x, wv_d, preferred_element_type=_f32)[:, None]


def _node_proj(x, w, a_src, a_dst):
    nb = 10
    blk = N // nb
    return pl.pallas_call(
        _node_proj_body,
        grid=(nb, NCORE),
        in_specs=[
            pl.BlockSpec((blk, F), lambda i, j: (i, 0)),
            pl.BlockSpec((F, H), lambda i, j: (0, 0)),
            pl.BlockSpec((H,), lambda i, j: (0,)),
            pl.BlockSpec((H,), lambda i, j: (0,)),
            pl.BlockSpec((F, HH), lambda i, j: (0, j)),
        ],
        out_specs=[
            pl.BlockSpec((blk, HH), lambda i, j: (j * nb + i, 0)),
            pl.BlockSpec((blk, 1), lambda i, j: (i, 0)),
            pl.BlockSpec((blk, 1), lambda i, j: (i, 0)),
        ],
        out_shape=[
            jax.ShapeDtypeStruct((2 * N, HH), _f32),
            jax.ShapeDtypeStruct((N, 1), _f32),
            jax.ShapeDtypeStruct((N, 1), _f32),
        ],
    )(x, w, a_src, a_dst, w)


def _edge_proj_body(ea_ref, we1_ref, ae1v_ref, we2_ref, ae2v_ref,
                    o1_ref, o2_ref):
    ea = ea_ref[...]
    w1 = jnp.dot(we1_ref[...], ae1v_ref[...], preferred_element_type=_f32)
    w2 = jnp.dot(we2_ref[...], ae2v_ref[...], preferred_element_type=_f32)
    o1_ref[...] = jnp.dot(ea, w1, preferred_element_type=_f32)[:, None]
    o2_ref[...] = jnp.dot(ea, w2, preferred_element_type=_f32)[:, None]


def _edge_proj(edge_attr, we1, a_e1, we2, a_e2):
    nb = 80
    blk = E // nb
    return pl.pallas_call(
        _edge_proj_body,
        grid=(nb,),
        in_specs=[
            pl.BlockSpec((blk, FE), lambda i: (i, 0)),
            pl.BlockSpec((FE, H), lambda i: (0, 0)),
            pl.BlockSpec((H,), lambda i: (0,)),
            pl.BlockSpec((FE, H), lambda i: (0, 0)),
            pl.BlockSpec((H,), lambda i: (0,)),
        ],
        out_specs=[
            pl.BlockSpec((blk, 1), lambda i: (i, 0)),
            pl.BlockSpec((blk, 1), lambda i: (i, 0)),
        ],
        out_shape=[
            jax.ShapeDtypeStruct((E, 1), _f32),
            jax.ShapeDtypeStruct((E, 1), _f32),
        ],
    )(edge_attr, we1, a_e1, we2, a_e2)


def _mid_proj_body(numlo_ref, numhi_ref, den_ref, b_ref, wfull_ref,
                   asrc_ref, adst_ref, whalf_ref, h_ref, ss_ref, sd_ref):
    num = jnp.concatenate([numlo_ref[...], numhi_ref[...]], axis=1)
    den = den_ref[...][0, 0][:, None]
    x = jnp.maximum(num / (den + 1e-16) + b_ref[...], 0.0)
    h_ref[...] = jnp.dot(x, whalf_ref[...], preferred_element_type=_f32)

    @pl.when(pl.program_id(1) == 0)
    def _():
        wv_s = jnp.dot(wfull_ref[...], asrc_ref[...],
                       preferred_element_type=_f32)
        wv_d = jnp.dot(wfull_ref[...], adst_ref[...],
                       preferred_element_type=_f32)
        ss_ref[...] = jnp.dot(x, wv_s, preferred_element_type=_f32)[:, None]
        sd_ref[...] = jnp.dot(x, wv_d, preferred_element_type=_f32)[:, None]


def _mid_proj(num, den, b, w, a_src, a_dst):
    nb = 10
    blk = N // nb
    return pl.pallas_call(
        _mid_proj_body,
        grid=(nb, NCORE),
        in_specs=[
            pl.BlockSpec((blk, HH), lambda i, j: (i, 0)),
            pl.BlockSpec((blk, HH), lambda i, j: (nb + i, 0)),
            pl.BlockSpec((1, 1, blk), lambda i, j: (i, 0, 0)),
            pl.BlockSpec((H,), lambda i, j: (0,)),
            pl.BlockSpec((H, H), lambda i, j: (0, 0)),
            pl.BlockSpec((H,), lambda i, j: (0,)),
            pl.BlockSpec((H,), lambda i, j: (0,)),
            pl.BlockSpec((H, HH), lambda i, j: (0, j)),
        ],
        out_specs=[
            pl.BlockSpec((blk, HH), lambda i, j: (j * nb + i, 0)),
            pl.BlockSpec((blk, 1), lambda i, j: (i, 0)),
            pl.BlockSpec((blk, 1), lambda i, j: (i, 0)),
        ],
        out_shape=[
            jax.ShapeDtypeStruct((2 * N, HH), _f32),
            jax.ShapeDtypeStruct((N, 1), _f32),
            jax.ShapeDtypeStruct((N, 1), _f32),
        ],
    )(num, num, den.reshape(nb, 1, blk), b, w, a_src, a_dst, w)


NSTEPS = -(-NCHUNKS // NSUB)
NP = -(-NSTEPS // 6) * 6


def _sc_message_body(h_hbm, ssrc_hbm, sdst_hbm, ae_hbm, src_hbm, dst_hbm,
                     num_hbm, den_hbm,
                     ssrc_v, sdst_v,
                     src0, dst0, ae0, ex0, src1, dst1, ae1, ex1,
                     src2, dst2, ae2, ex2, src3, dst3, ae3, ex3,
                     src4, dst4, ae4, ex4, src5, dst5, ae5, ex5,
                     rowsA, rowsB, rowsC,
                     zsem, stA, stB, gsA, gsB, wsA, wsB, acc_sh, den_sh):
    IDX = ((src0, dst0, ae0, ex0), (src1, dst1, ae1, ex1),
           (src2, dst2, ae2, ex2), (src3, dst3, ae3, ex3),
           (src4, dst4, ae4, ex4), (src5, dst5, ae5, ex5))
    ROWS = (rowsA, rowsB, rowsC)
    STSEM = (stA, stB)
    GSEM = (gsA, gsB)
    WSEM = (wsA, wsB)
    c = lax.axis_index("c")
    s = lax.axis_index("s")

    pltpu.sync_copy(ssrc_hbm, ssrc_v)
    pltpu.sync_copy(sdst_hbm, sdst_v)

    zeros = jnp.zeros((LN,), _f32)

    def _zero_rows(i, _):
        for k in range(HH // LN):
            rowsA[i, pl.ds(k * LN, LN)] = zeros
        return 0
    lax.fori_loop(0, LN, _zero_rows, 0)

    nzc = -(-(N // LN) // NSUB)

    def _zero_acc(i, _):
        ci = jnp.minimum(i * NSUB + s, N // LN - 1)
        pltpu.async_copy(rowsA.at[pl.ds(0, LN)],
                         acc_sh.at[pl.ds(ci * LN, LN)], zsem)
        pltpu.async_copy(rowsA.at[0].at[pl.ds(0, LN)],
                         den_sh.at[pl.ds(ci * LN, LN)], gsA)
        return 0
    lax.fori_loop(0, nzc, _zero_acc, 0)

    coff = c * N


    def _stage(cs, j, sem):
        ci = cs * NSUB + s
        base = jnp.minimum(ci, NCHUNKS - 1) * CK
        srcv, dstv, aev, _ = IDX[j]
        pltpu.async_copy(src_hbm.at[pl.ds(base, CK)], srcv, sem)
        pltpu.async_copy(dst_hbm.at[pl.ds(base, CK)], dstv, sem)
        pltpu.async_copy(ae_hbm.at[pl.ds(base, CK)], aev, sem)

    def _wait_stage(j, sem):
        srcv, dstv, aev, _ = IDX[j]
        pltpu.make_async_copy(src_hbm.at[pl.ds(0, CK)], srcv, sem).wait()
        pltpu.make_async_copy(dst_hbm.at[pl.ds(0, CK)], dstv, sem).wait()
        pltpu.make_async_copy(ae_hbm.at[pl.ds(0, CK)], aev, sem).wait()

    _stage(0, 0, STSEM[0])
    _stage(1, 1, STSEM[1])

    def _zero_wait(i, _):
        pltpu.make_async_copy(rowsA.at[pl.ds(0, LN)],
                              acc_sh.at[pl.ds(0, LN)], zsem).wait()
        pltpu.make_async_copy(rowsA.at[0].at[pl.ds(0, LN)],
                              den_sh.at[pl.ds(0, LN)], gsA).wait()
        return 0
    lax.fori_loop(0, nzc, _zero_wait, 0)

    plsc.subcore_barrier()

    def _wait_w(rowsv, dstv, exv, sem):
        pltpu.make_async_copy(rowsv, acc_sh.at[dstv], sem).wait()
        pltpu.make_async_copy(exv, den_sh.at[dstv], sem).wait()

    def _fix_gather(j, r, sem):
        srcv, dstv = IDX[j][0], IDX[j][1]
        raws, dsts = [], []
        for g in range(CK // LN):
            sl = pl.ds(g * LN, LN)
            sv = srcv[sl]
            raws.append(sv)
            dsts.append(dstv[sl])
            srcv[sl] = sv + coff
        pltpu.async_copy(h_hbm.at[srcv], ROWS[r], sem)
        return raws, dsts

    def _wait_gather(j, r, sem):
        pltpu.make_async_copy(h_hbm.at[IDX[j][0]], ROWS[r], sem).wait()

    def _ex_compute(ci, raws, dsts, aev, exv):
        vf = jnp.where(ci < NCHUNKS, 1.0, 0.0).astype(_f32)
        for j in range(CK // LN):
            sl = pl.ds(j * LN, LN)
            a = (plsc.load_gather(ssrc_v, [raws[j]])
                 + plsc.load_gather(sdst_v, [dsts[j]])
                 + aev[sl])
            a = jnp.maximum(a, 0.2 * a)
            exv[sl] = jnp.exp(a) * vf

    def _scale(rowsv, exv):
        def body(g, _):
            exg = exv[pl.ds(g * LN, LN)]
            for l in range(LN):
                e = exg[l]
                jj = g * LN + l
                for k in range(HH // LN):
                    slk = pl.ds(k * LN, LN)
                    rowsv[jj, slk] = rowsv[jj, slk] * e
            return 0
        lax.fori_loop(0, CK // LN, body, 0)

    def _scatter(rowsv, dstv, exv, sem):
        pltpu.async_copy(rowsv, acc_sh.at[dstv], sem, add=True)
        pltpu.async_copy(exv, den_sh.at[dstv], sem, add=True)

    def _chunk_issue(cs, j):
        raws, dsts = _fix_gather(j, j % 3, GSEM[j % 2])
        _ex_compute(cs * NSUB + s, raws, dsts, IDX[j][2], IDX[j][3])

    _wait_stage(0, STSEM[0])
    _chunk_issue(0, 0)

    def _body(k, _):
        for j in range(6):
            cs = 6 * k + j
            jn = (j + 1) % 6

            @pl.when(cs >= 2)
            def _():
                _wait_w(ROWS[(j - 2) % 3], IDX[(j - 2) % 6][1],
                        IDX[(j - 2) % 6][3], WSEM[j % 2])
            _stage(cs + 2, (j + 2) % 6, STSEM[j % 2])
            _wait_stage(jn, STSEM[jn % 2])
            _chunk_issue(cs + 1, jn)
            _wait_gather(j, j % 3, GSEM[j % 2])
            _scale(ROWS[j % 3], IDX[j][3])
            _scatter(ROWS[j % 3], IDX[j][1], IDX[j][3], WSEM[j % 2])
        return 0

    lax.fori_loop(0, NP // 6, _body, 0)

    _wait_w(ROWS[(NP - 2) % 3], IDX[(NP - 2) % 6][1], IDX[(NP - 2) % 6][3],
            WSEM[(NP - 2) % 2])
    _wait_w(ROWS[(NP - 1) % 3], IDX[(NP - 1) % 6][1], IDX[(NP - 1) % 6][3],
            WSEM[(NP - 1) % 2])
    _wait_gather(NP % 6, NP % 3, GSEM[NP % 2])
    _wait_stage((NP + 1) % 6, STSEM[(NP + 1) % 2])

    plsc.subcore_barrier()

    def _out_copy(i, _):
        ci = jnp.minimum(i * NSUB + s, N // LN - 1)
        pltpu.async_copy(acc_sh.at[pl.ds(ci * LN, LN)],
                         num_hbm.at[pl.ds(coff + ci * LN, LN)], zsem)
        return 0
    lax.fori_loop(0, nzc, _out_copy, 0)

    def _out_wait(i, _):
        pltpu.make_async_copy(acc_sh.at[pl.ds(0, LN)],
                              num_hbm.at[pl.ds(coff, LN)], zsem).wait()
        return 0
    lax.fori_loop(0, nzc, _out_wait, 0)

    @pl.when((s == 0) & (c == 0))
    def _():
        pltpu.sync_copy(den_sh, den_hbm)


_sc_message = functools.partial(
    pl.kernel,
    _sc_message_body,
    out_type=(
        jax.ShapeDtypeStruct((2 * N, HH), _f32),
        jax.ShapeDtypeStruct((N,), _f32),
    ),
    mesh=plsc.VectorSubcoreMesh(core_axis_name="c", subcore_axis_name="s",
                                num_cores=NCORE, num_subcores=NSUB),
    compiler_params=pltpu.CompilerParams(needs_layout_passes=False),
    scratch_types=[
        pltpu.VMEM((N,), _f32),
        pltpu.VMEM((N,), _f32),
    ] + [
        t
        for _set in range(6)
        for t in (pltpu.VMEM((CK,), jnp.int32),
                  pltpu.VMEM((CK,), jnp.int32),
                  pltpu.VMEM((CK,), _f32),
                  pltpu.VMEM((CK,), _f32))
    ] + [
        pltpu.VMEM((CK, HH), _f32),
        pltpu.VMEM((CK, HH), _f32),
        pltpu.VMEM((CK, HH), _f32),
        pltpu.SemaphoreType.DMA,
        pltpu.SemaphoreType.DMA,
        pltpu.SemaphoreType.DMA,
        pltpu.SemaphoreType.DMA,
        pltpu.SemaphoreType.DMA,
        pltpu.SemaphoreType.DMA,
        pltpu.SemaphoreType.DMA,
        pltpu.VMEM_SHARED((N, HH), _f32),
        pltpu.VMEM_SHARED((N,), _f32),
    ],
)()


NPT = 632
NPT_LAST = N - (NSUB - 1) * NPT


def _sc_pool_body(num_hbm, den_hbm, b2_hbm, batch_hbm,
                  gsum_hbm, gmax_hbm,
                  x_v, den_v, batch_v, b2_v, sum_v, max_v, mg_v, pool_sh):
    c = lax.axis_index("c")
    s = lax.axis_index("s")
    base = s * NPT
    is_last = s == NSUB - 1

    @pl.when(is_last)
    def _():
        pltpu.sync_copy(num_hbm.at[pl.ds(c * N + base, NPT_LAST)],
                        x_v.at[pl.ds(0, NPT_LAST)])
        pltpu.sync_copy(den_hbm.at[pl.ds(base, NPT_LAST)],
                        den_v.at[pl.ds(0, NPT_LAST)])
        pltpu.sync_copy(batch_hbm.at[pl.ds(base, NPT_LAST)],
                        batch_v.at[pl.ds(0, NPT_LAST)])

    @pl.when(jnp.logical_not(is_last))
    def _():
        pltpu.sync_copy(num_hbm.at[pl.ds(c * N + base, NPT)], x_v)
        pltpu.sync_copy(den_hbm.at[pl.ds(base, NPT)], den_v)
        pltpu.sync_copy(batch_hbm.at[pl.ds(base, NPT)], batch_v)

    pltpu.sync_copy(b2_hbm, b2_v)

    zeros = jnp.zeros((LN,), _f32)

    def _zero_acc(i, _):
        for k in range(HH // LN):
            sum_v[i, pl.ds(k * LN, LN)] = zeros
            max_v[i, pl.ds(k * LN, LN)] = zeros
        return 0
    lax.fori_loop(0, B, _zero_acc, 0)

    b2c = [b2_v[pl.ds(c * HH + k * LN, LN)] for k in range(HH // LN)]

    sz = jnp.where(is_last, NPT_LAST, NPT)
    ngroups = jnp.where(is_last, NPT_LAST // LN, NPT // LN)

    def _do_node(n, b, rec):
        for k in range(HH // LN):
            slk = pl.ds(k * LN, LN)
            xa = jnp.maximum(x_v[n, slk] * rec + b2c[k], 0.0)
            sum_v[b, slk] = sum_v[b, slk] + xa
            max_v[b, slk] = jnp.maximum(max_v[b, slk], xa)

    def _node(g, _):
        @pl.when(g < ngroups)
        def _():
            bv = batch_v[pl.ds(g * LN, LN)]
            dv = den_v[pl.ds(g * LN, LN)]
            recv = 1.0 / (dv + 1e-16)
            for l in range(LN):
                _do_node(g * LN + l, bv[l], recv[l])
        return 0
    lax.fori_loop(0, NPT // LN, _node, 0)

    bv = batch_v[pl.ds(sz - LN, LN)]
    dv = den_v[pl.ds(sz - LN, LN)]
    recv = 1.0 / (dv + 1e-16)
    for l in range(LN // 2, LN):
        _do_node(sz - LN + l, bv[l], recv[l])

    for acc_v, out_hbm, is_max in ((sum_v, gsum_hbm, False),
                                   (max_v, gmax_hbm, True)):
        pltpu.sync_copy(acc_v, pool_sh.at[s])
        plsc.subcore_barrier()

        @pl.when(s < NSUB // 2)
        def _():
            for t in range(NSUB):
                pltpu.sync_copy(pool_sh.at[t].at[pl.ds(s * SEG_PT, SEG_PT)],
                                mg_v.at[pl.ds(t * SEG_PT, SEG_PT)])

            def _merge_row(r, _):
                for k in range(HH // LN):
                    slk = pl.ds(k * LN, LN)
                    v = mg_v[r, slk]
                    for t in range(1, NSUB):
                        if is_max:
                            v = jnp.maximum(v, mg_v[t * SEG_PT + r, slk])
                        else:
                            v = v + mg_v[t * SEG_PT + r, slk]
                    sum_v[r, slk] = v
                return 0
            lax.fori_loop(0, SEG_PT, _merge_row, 0)
            pltpu.sync_copy(sum_v.at[pl.ds(0, SEG_PT)],
                            out_hbm.at[pl.ds(c * B + s * SEG_PT, SEG_PT)])
        plsc.subcore_barrier()


_sc_pool = functools.partial(
    pl.kernel,
    _sc_pool_body,
    out_type=(
        jax.ShapeDtypeStruct((NCORE * B, HH), _f32),
        jax.ShapeDtypeStruct((NCORE * B, HH), _f32),
    ),
    mesh=plsc.VectorSubcoreMesh(core_axis_name="c", subcore_axis_name="s",
                                num_cores=NCORE, num_subcores=NSUB),
    compiler_params=pltpu.CompilerParams(needs_layout_passes=False),
    scratch_types=[
        pltpu.VMEM((NPT, HH), _f32),
        pltpu.VMEM((NPT,), _f32),
        pltpu.VMEM((NPT,), jnp.int32),
        pltpu.VMEM((H,), _f32),
        pltpu.VMEM((B, HH), _f32),
        pltpu.VMEM((B, HH), _f32),
        pltpu.VMEM((NSUB * SEG_PT, HH), _f32),
        pltpu.VMEM_SHARED((NSUB, B, HH), _f32),
    ],
)()


def _final_body(gsum_ref, gmax_ref, batch_ref, clin_ref, wc1_ref, bc1_ref,
                wc2_ref, bc2_ref, wg1_ref, bg1_ref, wg2_ref, bg2_ref,
                wcls_ref, bcls_ref, o_ref):
    bt = batch_ref[...]
    eq = bt[:, None] == lax.broadcasted_iota(jnp.int32, (N, B), 1)
    cnt = jnp.sum(jnp.where(eq, 1.0, 0.0), axis=0)
    cnt = jnp.clip(cnt, 1.0, None)

    gsum = jnp.concatenate([gsum_ref[...][:B], gsum_ref[...][B:]], axis=1)
    gmax = jnp.concatenate([gmax_ref[...][:B], gmax_ref[...][B:]], axis=1)
    gmean = gsum / cnt[:, None]
    emb_g = jnp.concatenate([gmean, gmax], axis=1)

    hc = jnp.maximum(
        jnp.dot(clin_ref[...], wc1_ref[...], preferred_element_type=_f32)
        + bc1_ref[...], 0.0)
    emb_c = (jnp.dot(hc, wc2_ref[...], preferred_element_type=_f32)
             + bc2_ref[...])

    ng = jnp.sqrt(jnp.sum(emb_g * emb_g, axis=1, keepdims=True))
    emb_g = emb_g / jnp.maximum(ng, 1e-12)
    nc = jnp.sqrt(jnp.sum(emb_c * emb_c, axis=1, keepdims=True))
    emb_c = emb_c / jnp.maximum(nc, 1e-12)

    cat = jnp.concatenate([emb_g, emb_c], axis=1)
    g1 = jnp.maximum(
        jnp.dot(cat, wg1_ref[...], preferred_element_type=_f32)
        + bg1_ref[...], 0.0)
    gate = jax.nn.sigmoid(
        jnp.dot(g1, wg2_ref[...], preferred_element_type=_f32) + bg2_ref[...])
    comb = jnp.concatenate([emb_g, emb_c * gate], axis=-1)
    o_ref[...] = (jnp.dot(comb, wcls_ref[...], preferred_element_type=_f32)
                  + bcls_ref[...])


def _final(gsum, gmax, batch, clinical, wc1, bc1, wc2, bc2,
           wg1, bg1, wg2, bg2, wcls, bcls):
    return pl.pallas_call(
        _final_body,
        out_shape=jax.ShapeDtypeStruct((B, NCLS), _f32),
    )(gsum, gmax, batch, clinical, wc1, bc1, wc2, bc2,
      wg1, bg1, wg2, bg2, wcls, bcls)


def kernel(graph_data, edge_index, edge_attr, clinical_data, batch,
           W1, a_src1, a_dst1, We1, a_e1, b1,
           W2, a_src2, a_dst2, We2, a_e2, b2,
           Wc1, bc1, Wc2, bc2, Wg1, bg1, Wg2, bg2, Wcls, bcls):
    src = edge_index[0]
    dst = edge_index[1]

    h1, s1s, s1d = _node_proj(graph_data, W1, a_src1, a_dst1)
    ae1, ae2 = _edge_proj(edge_attr, We1, a_e1, We2, a_e2)

    num1, den1 = _sc_message(h1, jnp.squeeze(s1s, 1), jnp.squeeze(s1d, 1),
                             jnp.squeeze(ae1, 1), src, dst)

    h2, s2s, s2d = _mid_proj(num1, den1, b1, W2, a_src2, a_dst2)

    num2, den2 = _sc_message(h2, jnp.squeeze(s2s, 1), jnp.squeeze(s2d, 1),
                             jnp.squeeze(ae2, 1), src, dst)

    gsum, gmax = _sc_pool(num2, den2, b2, batch)

    return _final(gsum, gmax, batch, clinical_data,
                  Wc1, bc1, Wc2, bc2, Wg1, bg1, Wg2, bg2, Wcls, bcls)
